# Initial kernel scaffold; baseline (speedup 1.0000x reference)
import jax
import jax.numpy as jnp
from jax.experimental import pallas as pl


def kernel(x, edge_index, batch, W1, b1, W2, b2, Wh1, bh1, Wh2, bh2, Wo, bo):
    raise NotImplementedError("write your pallas kernel here")


# speedup vs baseline: 4.0780x; 4.0780x over previous
import functools

import jax
import jax.numpy as jnp
from jax import lax
from jax.experimental import pallas as pl
from jax.experimental.pallas import tpu as pltpu, tpu_sc as plsc

N_NODES = 10000
NPAD = 10240
E_EDGES = 160000
NC, NS = 2, 16
NW = NC * NS
EB = 128
EPAD = 163840
EPW = EPAD // NW
NBATCH = EPW // EB
ZR = NPAD // NS
FCH = 8
FW = 128
DUMMY = N_NODES

_mesh = plsc.VectorSubcoreMesh(
    core_axis_name="c", subcore_axis_name="s", num_cores=NC, num_subcores=NS)


def _zero_fill(buf, rows, width):
    z16 = jnp.zeros((16,), jnp.float32)

    @pl.loop(0, rows)
    def _(r):
        for c in range(width // 16):
            buf[r, pl.ds(c * 16, 16)] = z16


def _sc_scatter16(table, src, dst):
    @functools.partial(
        pl.kernel,
        out_type=jax.ShapeDtypeStruct((NC, NPAD, 16), jnp.float32),
        mesh=_mesh,
        scratch_types=[
            pltpu.VMEM_SHARED((NPAD, 16), jnp.float32),
            pltpu.VMEM((EB,), jnp.int32),
            pltpu.VMEM((EB,), jnp.int32),
            pltpu.VMEM((EB, 16), jnp.float32),
            pltpu.VMEM((EB, 16), jnp.float32),
            pltpu.SemaphoreType.DMA,
        ],
        compiler_params=pltpu.CompilerParams(use_tc_tiling_on_sc=False),
    )
    def k(table_h, src_h, dst_h, out_h, acc_sh, srcb, dstb, rowsb, zerob, sem):
        cid = lax.axis_index("c")
        sid = lax.axis_index("s")
        wid = sid * NC + cid
        _zero_fill(zerob, EB, 16)
        for t in range(ZR // EB):
            pltpu.sync_copy(zerob, acc_sh.at[pl.ds(sid * ZR + t * EB, EB)])
        plsc.subcore_barrier()

        @pl.loop(0, NBATCH)
        def _(j):
            off = wid * EPW + j * EB
            pltpu.sync_copy(src_h.at[pl.ds(off, EB)], srcb)
            pltpu.sync_copy(dst_h.at[pl.ds(off, EB)], dstb)
            pltpu.async_copy(table_h.at[srcb], rowsb, sem).wait()
            pltpu.sync_copy(rowsb, acc_sh.at[dstb], add=True)

        plsc.subcore_barrier()
        for t in range(ZR // EB):
            sl = pl.ds(sid * ZR + t * EB, EB)
            pltpu.sync_copy(acc_sh.at[sl], out_h.at[cid, sl])

    return k(table, src, dst)


def _sc_scatter128(vflat, srcoff, dst):
    @functools.partial(
        pl.kernel,
        out_type=jax.ShapeDtypeStruct((FCH, NC, NPAD, FW), jnp.float32),
        mesh=_mesh,
        scratch_types=[
            pltpu.VMEM_SHARED((NPAD, FW), jnp.float32),
            pltpu.VMEM((EB,), jnp.int32),
            pltpu.VMEM((EB,), jnp.int32),
            pltpu.VMEM((EB, FW), jnp.float32),
            pltpu.VMEM((EB, FW), jnp.float32),
            pltpu.SemaphoreType.DMA,
        ],
    )
    def k(v_h, srcoff_h, dst_h, out_h, acc_sh, srcb, dstb, rowsb, zerob, sem):
        cid = lax.axis_index("c")
        sid = lax.axis_index("s")
        wid = sid * NC + cid
        _zero_fill(zerob, EB, FW)

        @pl.loop(0, FCH)
        def _(ch):
            for t in range(ZR // EB):
                pltpu.sync_copy(zerob, acc_sh.at[pl.ds(sid * ZR + t * EB, EB)])
            plsc.subcore_barrier()

            @pl.loop(0, NBATCH)
            def _(j):
                off = wid * EPW + j * EB
                pltpu.sync_copy(srcoff_h.at[ch, pl.ds(off, EB)], srcb)
                pltpu.sync_copy(dst_h.at[pl.ds(off, EB)], dstb)
                pltpu.async_copy(v_h.at[srcb], rowsb, sem).wait()
                pltpu.sync_copy(rowsb, acc_sh.at[dstb], add=True)

            plsc.subcore_barrier()
            for t in range(ZR // EB):
                sl = pl.ds(sid * ZR + t * EB, EB)
                pltpu.sync_copy(acc_sh.at[sl], out_h.at[ch, cid, sl])
            plsc.subcore_barrier()

    return k(vflat, srcoff, dst)


def _tc_deg_finalize(h0, h1, xp):
    def body(h0_ref, h1_ref, xp_ref, dis_ref, v1_ref):
        deg = 1.0 + h0_ref[...] + h1_ref[...]
        dis = lax.rsqrt(deg)
        dis_ref[...] = dis
        v1_ref[...] = dis * xp_ref[...]

    return pl.pallas_call(
        body,
        out_shape=(jax.ShapeDtypeStruct((NPAD, 16), jnp.float32),
                   jax.ShapeDtypeStruct((NPAD, 16), jnp.float32)),
    )(h0, h1, xp)


BLK = 512
MGRID = NPAD // BLK


def _tc_mlp_mid(s0, s1, v1, dis, w1p, b1, w2):
    def body(s0_ref, s1_ref, v1_ref, dis_ref, w1_ref, b1_ref, w2_ref, out_ref):
        z = dis_ref[...] * (s0_ref[...] + s1_ref[...] + v1_ref[...])
        h = jnp.dot(z, w1_ref[...], preferred_element_type=jnp.float32)
        h = jnp.maximum(h + b1_ref[...], 0.0)
        u = jnp.dot(h, w2_ref[...], preferred_element_type=jnp.float32)
        v = dis_ref[...][:, 0:1] * u
        for c in range(FCH):
            out_ref[c] = v[:, c * FW:(c + 1) * FW]

    return pl.pallas_call(
        body,
        grid=(MGRID,),
        in_specs=[
            pl.BlockSpec((BLK, 16), lambda i: (i, 0)),
            pl.BlockSpec((BLK, 16), lambda i: (i, 0)),
            pl.BlockSpec((BLK, 16), lambda i: (i, 0)),
            pl.BlockSpec((BLK, 16), lambda i: (i, 0)),
            pl.BlockSpec((16, 2048), lambda i: (0, 0)),
            pl.BlockSpec((1, 2048), lambda i: (0, 0)),
            pl.BlockSpec((2048, 1024), lambda i: (0, 0)),
        ],
        out_specs=pl.BlockSpec((FCH, BLK, FW), lambda i: (0, i, 0)),
        out_shape=jax.ShapeDtypeStruct((FCH, NPAD, FW), jnp.float32),
        compiler_params=pltpu.CompilerParams(
            dimension_semantics=("arbitrary",)),
    )(s0, s1, v1, dis, w1p, b1, w2)


def _tc_pool_head(a0, a1, v, dis, b2, batchp, wh1, bh1, wh2, bh2, wop, bop):
    def body(a0_ref, a1_ref, v_ref, dis_ref, b2_ref, bat_ref,
             wh1_ref, bh1_ref, wh2_ref, bh2_ref, wo_ref, bo_ref,
             out_ref, psum_ref, cnt_ref):
        i = pl.program_id(0)

        @pl.when(i == 0)
        def _():
            psum_ref[...] = jnp.zeros_like(psum_ref)
            cnt_ref[...] = jnp.zeros_like(cnt_ref)

        oh = (bat_ref[...] == lax.broadcasted_iota(jnp.int32, (1, 32), 1))
        oh = oh.astype(jnp.float32)
        dnums = (((0,), (0,)), ((), ()))
        cnt_ref[...] += lax.dot_general(
            oh, jnp.ones((BLK, FW), jnp.float32), dnums,
            preferred_element_type=jnp.float32)
        d = dis_ref[...][:, 0:1]
        for c in range(FCH):
            y = d * (a0_ref[c] + a1_ref[c] + v_ref[c])
            y = jnp.maximum(y + b2_ref[...][:, c * FW:(c + 1) * FW], 0.0)
            psum_ref[:, c * FW:(c + 1) * FW] += lax.dot_general(
                oh, y, dnums, preferred_element_type=jnp.float32)

        @pl.when(i == MGRID - 1)
        def _():
            g = psum_ref[...] / jnp.maximum(cnt_ref[...][:, 0:1], 1.0)
            g = jnp.dot(g, wh1_ref[...], preferred_element_type=jnp.float32)
            g = jnp.maximum(g + bh1_ref[...], 0.0)
            g = jnp.dot(g, wh2_ref[...], preferred_element_type=jnp.float32)
            g = jnp.maximum(g + bh2_ref[...], 0.0)
            out_ref[...] = (jnp.dot(g, wo_ref[...],
                                    preferred_element_type=jnp.float32)
                            + bo_ref[...])

    return pl.pallas_call(
        body,
        grid=(MGRID,),
        in_specs=[
            pl.BlockSpec((FCH, BLK, FW), lambda i: (0, i, 0)),
            pl.BlockSpec((FCH, BLK, FW), lambda i: (0, i, 0)),
            pl.BlockSpec((FCH, BLK, FW), lambda i: (0, i, 0)),
            pl.BlockSpec((BLK, 16), lambda i: (i, 0)),
            pl.BlockSpec((1, 1024), lambda i: (0, 0)),
            pl.BlockSpec((BLK, 1), lambda i: (i, 0)),
            pl.BlockSpec((1024, 1024), lambda i: (0, 0)),
            pl.BlockSpec((1, 1024), lambda i: (0, 0)),
            pl.BlockSpec((1024, 512), lambda i: (0, 0)),
            pl.BlockSpec((1, 512), lambda i: (0, 0)),
            pl.BlockSpec((512, 128), lambda i: (0, 0)),
            pl.BlockSpec((1, 128), lambda i: (0, 0)),
        ],
        out_specs=pl.BlockSpec((32, 128), lambda i: (0, 0)),
        out_shape=jax.ShapeDtypeStruct((32, 128), jnp.float32),
        scratch_shapes=[
            pltpu.VMEM((32, 1024), jnp.float32),
            pltpu.VMEM((32, 128), jnp.float32),
        ],
        compiler_params=pltpu.CompilerParams(
            dimension_semantics=("arbitrary",)),
    )(a0, a1, v, dis, b2, batchp, wh1, bh1, wh2, bh2, wop, bop)


def kernel(x, edge_index, batch, W1, b1, W2, b2, Wh1, bh1, Wh2, bh2, Wo, bo):
    f32, i32 = jnp.float32, jnp.int32

    xp = jnp.zeros((NPAD, 16), f32).at[:N_NODES, :5].set(x)
    padidx = jnp.full((EPAD - E_EDGES,), DUMMY, i32)
    src = jnp.concatenate([edge_index[0], padidx])
    dst = jnp.concatenate([edge_index[1], padidx])
    srcoff = src[None, :] + (jnp.arange(FCH, dtype=i32) * NPAD)[:, None]
    ones16 = jnp.ones((NPAD, 16), f32)
    batchp = jnp.concatenate(
        [batch, jnp.full((NPAD - N_NODES,), 99, i32)]).reshape(NPAD, 1)
    w1p = jnp.zeros((16, 2048), f32).at[:5].set(W1)
    wop = jnp.zeros((512, 128), f32).at[:, :2].set(Wo)
    bop = jnp.zeros((1, 128), f32).at[0, :2].set(bo)

    hist = _sc_scatter16(ones16, dst, dst)
    dis, v1 = _tc_deg_finalize(hist[0], hist[1], xp)
    s1 = _sc_scatter16(v1, src, dst)
    v = _tc_mlp_mid(s1[0], s1[1], v1, dis, w1p, b1.reshape(1, -1), W2)
    acc = _sc_scatter128(v.reshape(FCH * NPAD, FW), srcoff, dst)
    out = _tc_pool_head(acc[:, 0], acc[:, 1], v, dis, b2.reshape(1, -1),
                        batchp, Wh1, bh1.reshape(1, -1), Wh2,
                        bh2.reshape(1, -1), wop, bop)
    return out[:, :2]


# speedup vs baseline: 5.0015x; 1.2265x over previous
import functools

import jax
import jax.numpy as jnp
from jax import lax
from jax.experimental import pallas as pl
from jax.experimental.pallas import tpu as pltpu, tpu_sc as plsc

N_NODES = 10000
NPAD = 10240
E_EDGES = 160000
NC, NS = 2, 16
NW = NC * NS
EB = 128
EPAD = 163840
EPW = EPAD // NW
NBATCH = EPW // EB
ZR = NPAD // NS
FCH = 8
FW = 128
DUMMY = N_NODES

_mesh = plsc.VectorSubcoreMesh(
    core_axis_name="c", subcore_axis_name="s", num_cores=NC, num_subcores=NS)


def _zero_fill(buf, rows, width):
    z16 = jnp.zeros((16,), jnp.float32)

    @pl.loop(0, rows)
    def _(r):
        for c in range(width // 16):
            buf[r, pl.ds(c * 16, 16)] = z16


NBUF = 2
LEAD = 1
ZEB = 32


def _edge_pipeline(table_h, acc_sh, srcb, dstb, rows, semg, sems):
    def wait_bytes(sem, slot):
        pltpu.make_async_copy(table_h.at[pl.ds(0, EB)], rows[slot], sem).wait()

    for b in range(LEAD):
        pltpu.async_copy(table_h.at[srcb.at[b]], rows[b], semg[b])

    @pl.loop(0, NBATCH // NBUF)
    def _(g):
        for b in range(NBUF):
            jb = g * NBUF + b
            wait_bytes(semg[b], b)
            pltpu.async_copy(rows[b], acc_sh.at[dstb.at[jb]], sems[b],
                             add=True)
            nslot = (b + LEAD) % NBUF
            nj = jb + LEAD

            @pl.when(nj < NBATCH)
            def _():
                @pl.when(nj >= NBUF)
                def _():
                    wait_bytes(sems[nslot], nslot)

                pltpu.async_copy(table_h.at[srcb.at[nj]], rows[nslot],
                                 semg[nslot])

    for b in range(NBUF):
        wait_bytes(sems[b], b)


def _sc_scatter16(table, src2d, dst2d):
    @functools.partial(
        pl.kernel,
        out_type=jax.ShapeDtypeStruct((NC, NPAD, 16), jnp.float32),
        mesh=_mesh,
        scratch_types=[
            pltpu.VMEM_SHARED((NPAD, 16), jnp.float32),
            pltpu.VMEM((NBATCH, EB), jnp.int32),
            pltpu.VMEM((NBATCH, EB), jnp.int32),
            [pltpu.VMEM((EB, 16), jnp.float32)] * NBUF,
            pltpu.VMEM((ZEB, 16), jnp.float32),
            [pltpu.SemaphoreType.DMA] * NBUF,
            [pltpu.SemaphoreType.DMA] * NBUF,
        ],
        compiler_params=pltpu.CompilerParams(use_tc_tiling_on_sc=False),
    )
    def k(table_h, src_h, dst_h, out_h, acc_sh, srcb, dstb, rows, zerob,
          semg, sems):
        cid = lax.axis_index("c")
        sid = lax.axis_index("s")
        wid = sid * NC + cid
        _zero_fill(zerob, ZEB, 16)
        pltpu.sync_copy(src_h.at[wid, :, :], srcb)
        pltpu.sync_copy(dst_h.at[wid, :, :], dstb)
        for t in range(ZR // ZEB):
            pltpu.sync_copy(zerob, acc_sh.at[pl.ds(sid * ZR + t * ZEB, ZEB)])
        plsc.subcore_barrier()
        _edge_pipeline(table_h, acc_sh, srcb, dstb, rows, semg, sems)
        plsc.subcore_barrier()
        for t in range(ZR // EB):
            sl = pl.ds(sid * ZR + t * EB, EB)
            pltpu.sync_copy(acc_sh.at[sl], out_h.at[cid, sl])

    return k(table, src2d, dst2d)


def _sc_scatter128(vflat, srcoff, dst2d):
    @functools.partial(
        pl.kernel,
        out_type=jax.ShapeDtypeStruct((FCH, NC, NPAD, FW), jnp.float32),
        mesh=_mesh,
        scratch_types=[
            pltpu.VMEM_SHARED((NPAD, FW), jnp.float32),
            pltpu.VMEM((NBATCH, EB), jnp.int32),
            pltpu.VMEM((NBATCH, EB), jnp.int32),
            [pltpu.VMEM((EB, FW), jnp.float32)] * NBUF,
            pltpu.VMEM((ZEB, FW), jnp.float32),
            [pltpu.SemaphoreType.DMA] * NBUF,
            [pltpu.SemaphoreType.DMA] * NBUF,
        ],
    )
    def k(v_h, srcoff_h, dst_h, out_h, acc_sh, srcb, dstb, rows, zerob,
          semg, sems):
        cid = lax.axis_index("c")
        sid = lax.axis_index("s")
        wid = sid * NC + cid
        _zero_fill(zerob, ZEB, FW)
        pltpu.sync_copy(dst_h.at[wid, :, :], dstb)

        @pl.loop(0, FCH)
        def _(ch):
            pltpu.sync_copy(srcoff_h.at[ch, wid, :, :], srcb)
            for t in range(ZR // ZEB):
                pltpu.sync_copy(zerob, acc_sh.at[pl.ds(sid * ZR + t * ZEB, ZEB)])
            plsc.subcore_barrier()
            _edge_pipeline(v_h, acc_sh, srcb, dstb, rows, semg, sems)
            plsc.subcore_barrier()
            for t in range(ZR // EB):
                sl = pl.ds(sid * ZR + t * EB, EB)
                pltpu.sync_copy(acc_sh.at[sl], out_h.at[ch, cid, sl])
            plsc.subcore_barrier()

    return k(vflat, srcoff, dst2d)


def _tc_deg_finalize(h0, h1, xp):
    def body(h0_ref, h1_ref, xp_ref, dis_ref, v1_ref):
        deg = 1.0 + h0_ref[...] + h1_ref[...]
        dis = lax.rsqrt(deg)
        dis_ref[...] = dis
        v1_ref[...] = dis * xp_ref[...]

    return pl.pallas_call(
        body,
        out_shape=(jax.ShapeDtypeStruct((NPAD, 16), jnp.float32),
                   jax.ShapeDtypeStruct((NPAD, 16), jnp.float32)),
    )(h0, h1, xp)


BLK = 512
MGRID = NPAD // BLK


def _tc_mlp_mid(s0, s1, v1, dis, w1p, b1, w2):
    def body(s0_ref, s1_ref, v1_ref, dis_ref, w1_ref, b1_ref, w2_ref, out_ref):
        z = dis_ref[...] * (s0_ref[...] + s1_ref[...] + v1_ref[...])
        h = jnp.dot(z, w1_ref[...], preferred_element_type=jnp.float32)
        h = jnp.maximum(h + b1_ref[...], 0.0)
        u = jnp.dot(h, w2_ref[...], preferred_element_type=jnp.float32)
        v = dis_ref[...][:, 0:1] * u
        for c in range(FCH):
            out_ref[c] = v[:, c * FW:(c + 1) * FW]

    return pl.pallas_call(
        body,
        grid=(MGRID,),
        in_specs=[
            pl.BlockSpec((BLK, 16), lambda i: (i, 0)),
            pl.BlockSpec((BLK, 16), lambda i: (i, 0)),
            pl.BlockSpec((BLK, 16), lambda i: (i, 0)),
            pl.BlockSpec((BLK, 16), lambda i: (i, 0)),
            pl.BlockSpec((16, 2048), lambda i: (0, 0)),
            pl.BlockSpec((1, 2048), lambda i: (0, 0)),
            pl.BlockSpec((2048, 1024), lambda i: (0, 0)),
        ],
        out_specs=pl.BlockSpec((FCH, BLK, FW), lambda i: (0, i, 0)),
        out_shape=jax.ShapeDtypeStruct((FCH, NPAD, FW), jnp.float32),
        compiler_params=pltpu.CompilerParams(
            dimension_semantics=("arbitrary",)),
    )(s0, s1, v1, dis, w1p, b1, w2)


def _tc_pool_head(a0, a1, v, dis, b2, batchp, wh1, bh1, wh2, bh2, wop, bop):
    def body(a0_ref, a1_ref, v_ref, dis_ref, b2_ref, bat_ref,
             wh1_ref, bh1_ref, wh2_ref, bh2_ref, wo_ref, bo_ref,
             out_ref, psum_ref, cnt_ref):
        i = pl.program_id(0)

        @pl.when(i == 0)
        def _():
            psum_ref[...] = jnp.zeros_like(psum_ref)
            cnt_ref[...] = jnp.zeros_like(cnt_ref)

        oh = (bat_ref[...] == lax.broadcasted_iota(jnp.int32, (1, 32), 1))
        oh = oh.astype(jnp.float32)
        dnums = (((0,), (0,)), ((), ()))
        cnt_ref[...] += lax.dot_general(
            oh, jnp.ones((BLK, FW), jnp.float32), dnums,
            preferred_element_type=jnp.float32)
        d = dis_ref[...][:, 0:1]
        for c in range(FCH):
            y = d * (a0_ref[c] + a1_ref[c] + v_ref[c])
            y = jnp.maximum(y + b2_ref[...][:, c * FW:(c + 1) * FW], 0.0)
            psum_ref[:, c * FW:(c + 1) * FW] += lax.dot_general(
                oh, y, dnums, preferred_element_type=jnp.float32)

        @pl.when(i == MGRID - 1)
        def _():
            g = psum_ref[...] / jnp.maximum(cnt_ref[...][:, 0:1], 1.0)
            g = jnp.dot(g, wh1_ref[...], preferred_element_type=jnp.float32)
            g = jnp.maximum(g + bh1_ref[...], 0.0)
            g = jnp.dot(g, wh2_ref[...], preferred_element_type=jnp.float32)
            g = jnp.maximum(g + bh2_ref[...], 0.0)
            out_ref[...] = (jnp.dot(g, wo_ref[...],
                                    preferred_element_type=jnp.float32)
                            + bo_ref[...])

    return pl.pallas_call(
        body,
        grid=(MGRID,),
        in_specs=[
            pl.BlockSpec((FCH, BLK, FW), lambda i: (0, i, 0)),
            pl.BlockSpec((FCH, BLK, FW), lambda i: (0, i, 0)),
            pl.BlockSpec((FCH, BLK, FW), lambda i: (0, i, 0)),
            pl.BlockSpec((BLK, 16), lambda i: (i, 0)),
            pl.BlockSpec((1, 1024), lambda i: (0, 0)),
            pl.BlockSpec((BLK, 1), lambda i: (i, 0)),
            pl.BlockSpec((1024, 1024), lambda i: (0, 0)),
            pl.BlockSpec((1, 1024), lambda i: (0, 0)),
            pl.BlockSpec((1024, 512), lambda i: (0, 0)),
            pl.BlockSpec((1, 512), lambda i: (0, 0)),
            pl.BlockSpec((512, 128), lambda i: (0, 0)),
            pl.BlockSpec((1, 128), lambda i: (0, 0)),
        ],
        out_specs=pl.BlockSpec((32, 128), lambda i: (0, 0)),
        out_shape=jax.ShapeDtypeStruct((32, 128), jnp.float32),
        scratch_shapes=[
            pltpu.VMEM((32, 1024), jnp.float32),
            pltpu.VMEM((32, 128), jnp.float32),
        ],
        compiler_params=pltpu.CompilerParams(
            dimension_semantics=("arbitrary",)),
    )(a0, a1, v, dis, b2, batchp, wh1, bh1, wh2, bh2, wop, bop)


def kernel(x, edge_index, batch, W1, b1, W2, b2, Wh1, bh1, Wh2, bh2, Wo, bo):
    f32, i32 = jnp.float32, jnp.int32

    xp = jnp.zeros((NPAD, 16), f32).at[:N_NODES, :5].set(x)
    padidx = jnp.full((EPAD - E_EDGES,), DUMMY, i32)
    src = jnp.concatenate([edge_index[0], padidx])
    dst = jnp.concatenate([edge_index[1], padidx])
    src2d = src.reshape(NW, NBATCH, EB)
    dst2d = dst.reshape(NW, NBATCH, EB)
    srcoff = (src[None, :]
              + (jnp.arange(FCH, dtype=i32) * NPAD)[:, None]).reshape(
                  FCH, NW, NBATCH, EB)
    ones16 = jnp.ones((NPAD, 16), f32)
    batchp = jnp.concatenate(
        [batch, jnp.full((NPAD - N_NODES,), 99, i32)]).reshape(NPAD, 1)
    w1p = jnp.zeros((16, 2048), f32).at[:5].set(W1)
    wop = jnp.zeros((512, 128), f32).at[:, :2].set(Wo)
    bop = jnp.zeros((1, 128), f32).at[0, :2].set(bo)

    hist = _sc_scatter16(ones16, dst2d, dst2d)
    dis, v1 = _tc_deg_finalize(hist[0], hist[1], xp)
    s1 = _sc_scatter16(v1, src2d, dst2d)
    v = _tc_mlp_mid(s1[0], s1[1], v1, dis, w1p, b1.reshape(1, -1), W2)
    acc = _sc_scatter128(v.reshape(FCH * NPAD, FW), srcoff, dst2d)
    out = _tc_pool_head(acc[:, 0], acc[:, 1], v, dis, b2.reshape(1, -1),
                        batchp, Wh1, bh1.reshape(1, -1), Wh2,
                        bh2.reshape(1, -1), wop, bop)
    return out[:, :2]


# speedup vs baseline: 12.9003x; 2.5793x over previous
import functools

import jax
import jax.numpy as jnp
from jax import lax
from jax.experimental import pallas as pl
from jax.experimental.pallas import tpu as pltpu, tpu_sc as plsc

N_NODES = 10000
NPAD = 10240
E_EDGES = 160000
NC, NS = 2, 16
NW = NC * NS
EB = 128
EPAD = 163840
EPW = EPAD // NW
NBATCH = EPW // EB
ZR = NPAD // NS
FCH = 8
FW = 128
DUMMY = N_NODES

_mesh = plsc.VectorSubcoreMesh(
    core_axis_name="c", subcore_axis_name="s", num_cores=NC, num_subcores=NS)


def _zero_fill(buf, rows, width):
    z16 = jnp.zeros((16,), jnp.float32)

    @pl.loop(0, rows)
    def _(r):
        for c in range(width // 16):
            buf[r, pl.ds(c * 16, 16)] = z16


NBUF = 2
LEAD = 1
ZEB = 32


def _edge_pipeline(table_h, acc_sh, srcb, dstb, rows, semg, sems):
    def wait_bytes(sem, slot):
        pltpu.make_async_copy(table_h.at[pl.ds(0, EB)], rows[slot], sem).wait()

    for b in range(LEAD):
        pltpu.async_copy(table_h.at[srcb.at[b]], rows[b], semg[b])

    @pl.loop(0, NBATCH // NBUF)
    def _(g):
        for b in range(NBUF):
            jb = g * NBUF + b
            wait_bytes(semg[b], b)
            pltpu.async_copy(rows[b], acc_sh.at[dstb.at[jb]], sems[b],
                             add=True)
            nslot = (b + LEAD) % NBUF
            nj = jb + LEAD

            @pl.when(nj < NBATCH)
            def _():
                @pl.when(nj >= NBUF)
                def _():
                    wait_bytes(sems[nslot], nslot)

                pltpu.async_copy(table_h.at[srcb.at[nj]], rows[nslot],
                                 semg[nslot])

    for b in range(NBUF):
        wait_bytes(sems[b], b)


def _sc_scatter16(table, src2d, dst2d):
    @functools.partial(
        pl.kernel,
        out_type=jax.ShapeDtypeStruct((NC, NPAD, 16), jnp.float32),
        mesh=_mesh,
        scratch_types=[
            pltpu.VMEM_SHARED((NPAD, 16), jnp.float32),
            pltpu.VMEM((NBATCH, EB), jnp.int32),
            pltpu.VMEM((NBATCH, EB), jnp.int32),
            [pltpu.VMEM((EB, 16), jnp.float32)] * NBUF,
            pltpu.VMEM((ZEB, 16), jnp.float32),
            [pltpu.SemaphoreType.DMA] * NBUF,
            [pltpu.SemaphoreType.DMA] * NBUF,
        ],
        compiler_params=pltpu.CompilerParams(use_tc_tiling_on_sc=False),
    )
    def k(table_h, src_h, dst_h, out_h, acc_sh, srcb, dstb, rows, zerob,
          semg, sems):
        cid = lax.axis_index("c")
        sid = lax.axis_index("s")
        wid = sid * NC + cid
        _zero_fill(zerob, ZEB, 16)
        pltpu.sync_copy(src_h.at[wid, :, :], srcb)
        pltpu.sync_copy(dst_h.at[wid, :, :], dstb)
        for t in range(ZR // ZEB):
            pltpu.sync_copy(zerob, acc_sh.at[pl.ds(sid * ZR + t * ZEB, ZEB)])
        plsc.subcore_barrier()
        _edge_pipeline(table_h, acc_sh, srcb, dstb, rows, semg, sems)
        plsc.subcore_barrier()
        for t in range(ZR // EB):
            sl = pl.ds(sid * ZR + t * EB, EB)
            pltpu.sync_copy(acc_sh.at[sl], out_h.at[cid, sl])

    return k(table, src2d, dst2d)


def _sc_scatter128(vflat, srcoff, dst2d):
    @functools.partial(
        pl.kernel,
        out_type=jax.ShapeDtypeStruct((FCH, NC, NPAD, FW), jnp.float32),
        mesh=_mesh,
        scratch_types=[
            pltpu.VMEM_SHARED((NPAD, FW), jnp.float32),
            pltpu.VMEM((NBATCH, EB), jnp.int32),
            pltpu.VMEM((NBATCH, EB), jnp.int32),
            [pltpu.VMEM((EB, FW), jnp.float32)] * NBUF,
            pltpu.VMEM((ZEB, FW), jnp.float32),
            [pltpu.SemaphoreType.DMA] * NBUF,
            [pltpu.SemaphoreType.DMA] * NBUF,
        ],
    )
    def k(v_h, srcoff_h, dst_h, out_h, acc_sh, srcb, dstb, rows, zerob,
          semg, sems):
        cid = lax.axis_index("c")
        sid = lax.axis_index("s")
        wid = sid * NC + cid
        _zero_fill(zerob, ZEB, FW)
        pltpu.sync_copy(dst_h.at[wid, :, :], dstb)

        @pl.loop(0, FCH)
        def _(ch):
            pltpu.sync_copy(srcoff_h.at[ch, wid, :, :], srcb)
            for t in range(ZR // ZEB):
                pltpu.sync_copy(zerob, acc_sh.at[pl.ds(sid * ZR + t * ZEB, ZEB)])
            plsc.subcore_barrier()
            _edge_pipeline(v_h, acc_sh, srcb, dstb, rows, semg, sems)
            plsc.subcore_barrier()
            for t in range(ZR // EB):
                sl = pl.ds(sid * ZR + t * EB, EB)
                pltpu.sync_copy(acc_sh.at[sl], out_h.at[ch, cid, sl])
            plsc.subcore_barrier()

    return k(vflat, srcoff, dst2d)


def _tc_deg_finalize(h0, h1, xp):
    def body(h0_ref, h1_ref, xp_ref, dis_ref, v1_ref):
        deg = 1.0 + h0_ref[...] + h1_ref[...]
        dis = lax.rsqrt(deg)
        dis_ref[...] = dis
        v1_ref[...] = dis * xp_ref[...]

    return pl.pallas_call(
        body,
        out_shape=(jax.ShapeDtypeStruct((NPAD, 16), jnp.float32),
                   jax.ShapeDtypeStruct((NPAD, 16), jnp.float32)),
    )(h0, h1, xp)


BLK = 512
MGRID = NPAD // BLK


def _tc_mlp_mid(s0, s1, v1, dis, w1p, b1, w2):
    def body(s0_ref, s1_ref, v1_ref, dis_ref, w1_ref, b1_ref, w2_ref, out_ref):
        z = dis_ref[...] * (s0_ref[...] + s1_ref[...] + v1_ref[...])
        h = jnp.dot(z, w1_ref[...], preferred_element_type=jnp.float32)
        h = jnp.maximum(h + b1_ref[...], 0.0)
        u = jnp.dot(h, w2_ref[...], preferred_element_type=jnp.float32)
        v = dis_ref[...][:, 0:1] * u
        for c in range(FCH):
            out_ref[c] = v[:, c * FW:(c + 1) * FW]

    return pl.pallas_call(
        body,
        grid=(MGRID,),
        in_specs=[
            pl.BlockSpec((BLK, 16), lambda i: (i, 0)),
            pl.BlockSpec((BLK, 16), lambda i: (i, 0)),
            pl.BlockSpec((BLK, 16), lambda i: (i, 0)),
            pl.BlockSpec((BLK, 16), lambda i: (i, 0)),
            pl.BlockSpec((16, 2048), lambda i: (0, 0)),
            pl.BlockSpec((1, 2048), lambda i: (0, 0)),
            pl.BlockSpec((2048, 1024), lambda i: (0, 0)),
        ],
        out_specs=pl.BlockSpec((FCH, BLK, FW), lambda i: (0, i, 0)),
        out_shape=jax.ShapeDtypeStruct((FCH, NPAD, FW), jnp.float32),
        compiler_params=pltpu.CompilerParams(
            dimension_semantics=("arbitrary",)),
    )(s0, s1, v1, dis, w1p, b1, w2)


def _tc_pool_head(a0, a1, v, dis, b2, batchp, wh1, bh1, wh2, bh2, wop, bop):
    def body(a0_ref, a1_ref, v_ref, dis_ref, b2_ref, bat_ref,
             wh1_ref, bh1_ref, wh2_ref, bh2_ref, wo_ref, bo_ref,
             out_ref, psum_ref, cnt_ref):
        i = pl.program_id(0)

        @pl.when(i == 0)
        def _():
            psum_ref[...] = jnp.zeros_like(psum_ref)
            cnt_ref[...] = jnp.zeros_like(cnt_ref)

        oh = (bat_ref[...] == lax.broadcasted_iota(jnp.int32, (1, 32), 1))
        oh = oh.astype(jnp.float32)
        dnums = (((0,), (0,)), ((), ()))
        cnt_ref[...] += lax.dot_general(
            oh, jnp.ones((BLK, FW), jnp.float32), dnums,
            preferred_element_type=jnp.float32)
        d = dis_ref[...][:, 0:1]
        for c in range(FCH):
            y = d * (a0_ref[c] + a1_ref[c] + v_ref[c])
            y = jnp.maximum(y + b2_ref[...][:, c * FW:(c + 1) * FW], 0.0)
            psum_ref[:, c * FW:(c + 1) * FW] += lax.dot_general(
                oh, y, dnums, preferred_element_type=jnp.float32)

        @pl.when(i == MGRID - 1)
        def _():
            g = psum_ref[...] / jnp.maximum(cnt_ref[...][:, 0:1], 1.0)
            g = jnp.dot(g, wh1_ref[...], preferred_element_type=jnp.float32)
            g = jnp.maximum(g + bh1_ref[...], 0.0)
            g = jnp.dot(g, wh2_ref[...], preferred_element_type=jnp.float32)
            g = jnp.maximum(g + bh2_ref[...], 0.0)
            out_ref[...] = (jnp.dot(g, wo_ref[...],
                                    preferred_element_type=jnp.float32)
                            + bo_ref[...])

    return pl.pallas_call(
        body,
        grid=(MGRID,),
        in_specs=[
            pl.BlockSpec((FCH, BLK, FW), lambda i: (0, i, 0)),
            pl.BlockSpec((FCH, BLK, FW), lambda i: (0, i, 0)),
            pl.BlockSpec((FCH, BLK, FW), lambda i: (0, i, 0)),
            pl.BlockSpec((BLK, 16), lambda i: (i, 0)),
            pl.BlockSpec((1, 1024), lambda i: (0, 0)),
            pl.BlockSpec((BLK, 1), lambda i: (i, 0)),
            pl.BlockSpec((1024, 1024), lambda i: (0, 0)),
            pl.BlockSpec((1, 1024), lambda i: (0, 0)),
            pl.BlockSpec((1024, 512), lambda i: (0, 0)),
            pl.BlockSpec((1, 512), lambda i: (0, 0)),
            pl.BlockSpec((512, 128), lambda i: (0, 0)),
            pl.BlockSpec((1, 128), lambda i: (0, 0)),
        ],
        out_specs=pl.BlockSpec((32, 128), lambda i: (0, 0)),
        out_shape=jax.ShapeDtypeStruct((32, 128), jnp.float32),
        scratch_shapes=[
            pltpu.VMEM((32, 1024), jnp.float32),
            pltpu.VMEM((32, 128), jnp.float32),
        ],
        compiler_params=pltpu.CompilerParams(
            dimension_semantics=("arbitrary",)),
    )(a0, a1, v, dis, b2, batchp, wh1, bh1, wh2, bh2, wop, bop)


def kernel(x, edge_index, batch, W1, b1, W2, b2, Wh1, bh1, Wh2, bh2, Wo, bo):
    f32, i32 = jnp.float32, jnp.int32

    xp = jnp.zeros((NPAD, 16), f32).at[:N_NODES, :5].set(x)
    npad_e = EPAD - E_EDGES
    ppw = npad_e // NW
    padidx = (DUMMY + jnp.arange(npad_e, dtype=i32) % (NPAD - N_NODES)
              ).reshape(NW, ppw)
    src2d = jnp.concatenate(
        [edge_index[0].reshape(NW, E_EDGES // NW), padidx], axis=1
    ).reshape(NW, NBATCH, EB)
    dst2d = jnp.concatenate(
        [edge_index[1].reshape(NW, E_EDGES // NW), padidx], axis=1
    ).reshape(NW, NBATCH, EB)
    srcoff = (src2d[None]
              + (jnp.arange(FCH, dtype=i32) * NPAD)[:, None, None, None])
    ones16 = jnp.ones((NPAD, 16), f32)
    batchp = jnp.concatenate(
        [batch, jnp.full((NPAD - N_NODES,), 99, i32)]).reshape(NPAD, 1)
    w1p = jnp.zeros((16, 2048), f32).at[:5].set(W1)
    wop = jnp.zeros((512, 128), f32).at[:, :2].set(Wo)
    bop = jnp.zeros((1, 128), f32).at[0, :2].set(bo)

    hist = _sc_scatter16(ones16, dst2d, dst2d)
    dis, v1 = _tc_deg_finalize(hist[0], hist[1], xp)
    s1 = _sc_scatter16(v1, src2d, dst2d)
    v = _tc_mlp_mid(s1[0], s1[1], v1, dis, w1p, b1.reshape(1, -1), W2)
    acc = _sc_scatter128(v.reshape(FCH * NPAD, FW), srcoff, dst2d)
    out = _tc_pool_head(acc[:, 0], acc[:, 1], v, dis, b2.reshape(1, -1),
                        batchp, Wh1, bh1.reshape(1, -1), Wh2,
                        bh2.reshape(1, -1), wop, bop)
    return out[:, :2]


# speedup vs baseline: 13.7656x; 1.0671x over previous
import functools

import jax
import jax.numpy as jnp
from jax import lax
from jax.experimental import pallas as pl
from jax.experimental.pallas import tpu as pltpu, tpu_sc as plsc

N_NODES = 10000
NPAD = 10240
E_EDGES = 160000
NC, NS = 2, 16
NW = NC * NS
EB = 128
EPAD = 163840
EPW = EPAD // NW
NBATCH = EPW // EB
ZR = NPAD // NS
FCH = 8
FW = 128
DUMMY = N_NODES

_mesh = plsc.VectorSubcoreMesh(
    core_axis_name="c", subcore_axis_name="s", num_cores=NC, num_subcores=NS)


def _zero_fill(buf, rows, width):
    z16 = jnp.zeros((16,), jnp.float32)

    @pl.loop(0, rows)
    def _(r):
        for c in range(width // 16):
            buf[r, pl.ds(c * 16, 16)] = z16


NBUF = 2
LEAD = 1
ZEB = 32


def _edge_pipeline(table_h, acc_sh, srcb, dstb, rows, semg, sems):
    def wait_bytes(sem, slot):
        pltpu.make_async_copy(table_h.at[pl.ds(0, EB)], rows[slot], sem).wait()

    for b in range(LEAD):
        pltpu.async_copy(table_h.at[srcb.at[b]], rows[b], semg[b])

    @pl.loop(0, NBATCH // NBUF)
    def _(g):
        for b in range(NBUF):
            jb = g * NBUF + b
            wait_bytes(semg[b], b)
            pltpu.async_copy(rows[b], acc_sh.at[dstb.at[jb]], sems[b],
                             add=True)
            nslot = (b + LEAD) % NBUF
            nj = jb + LEAD

            @pl.when(nj < NBATCH)
            def _():
                @pl.when(nj >= NBUF)
                def _():
                    wait_bytes(sems[nslot], nslot)

                pltpu.async_copy(table_h.at[srcb.at[nj]], rows[nslot],
                                 semg[nslot])

    for b in range(NBUF):
        wait_bytes(sems[b], b)


def _sc_scatter16(table, src2d, dst2d):
    @functools.partial(
        pl.kernel,
        out_type=jax.ShapeDtypeStruct((NC, NPAD, 16), jnp.float32),
        mesh=_mesh,
        scratch_types=[
            pltpu.VMEM_SHARED((NPAD, 16), jnp.float32),
            pltpu.VMEM((NBATCH, EB), jnp.int32),
            pltpu.VMEM((NBATCH, EB), jnp.int32),
            [pltpu.VMEM((EB, 16), jnp.float32)] * NBUF,
            pltpu.VMEM((ZEB, 16), jnp.float32),
            [pltpu.SemaphoreType.DMA] * NBUF,
            [pltpu.SemaphoreType.DMA] * NBUF,
        ],
        compiler_params=pltpu.CompilerParams(use_tc_tiling_on_sc=False),
    )
    def k(table_h, src_h, dst_h, out_h, acc_sh, srcb, dstb, rows, zerob,
          semg, sems):
        cid = lax.axis_index("c")
        sid = lax.axis_index("s")
        wid = sid * NC + cid
        _zero_fill(zerob, ZEB, 16)
        pltpu.sync_copy(src_h.at[wid, :, :], srcb)
        pltpu.sync_copy(dst_h.at[wid, :, :], dstb)
        for t in range(ZR // ZEB):
            pltpu.sync_copy(zerob, acc_sh.at[pl.ds(sid * ZR + t * ZEB, ZEB)])
        plsc.subcore_barrier()
        _edge_pipeline(table_h, acc_sh, srcb, dstb, rows, semg, sems)
        plsc.subcore_barrier()
        for t in range(ZR // EB):
            sl = pl.ds(sid * ZR + t * EB, EB)
            pltpu.sync_copy(acc_sh.at[sl], out_h.at[cid, sl])

    return k(table, src2d, dst2d)


def _sc_scatter128(vflat, srcoff, dst2d):
    @functools.partial(
        pl.kernel,
        out_type=jax.ShapeDtypeStruct((FCH, NC, NPAD, FW), jnp.float32),
        mesh=_mesh,
        scratch_types=[
            pltpu.VMEM_SHARED((NPAD, FW), jnp.float32),
            pltpu.VMEM((NBATCH, EB), jnp.int32),
            pltpu.VMEM((NBATCH, EB), jnp.int32),
            [pltpu.VMEM((EB, FW), jnp.float32)] * NBUF,
            pltpu.VMEM((ZEB, FW), jnp.float32),
            [pltpu.SemaphoreType.DMA] * NBUF,
            [pltpu.SemaphoreType.DMA] * NBUF,
        ],
    )
    def k(v_h, srcoff_h, dst_h, out_h, acc_sh, srcb, dstb, rows, zerob,
          semg, sems):
        cid = lax.axis_index("c")
        sid = lax.axis_index("s")
        wid = sid * NC + cid
        _zero_fill(zerob, ZEB, FW)
        pltpu.sync_copy(dst_h.at[wid, :, :], dstb)

        @pl.loop(0, FCH)
        def _(ch):
            pltpu.sync_copy(srcoff_h.at[ch, wid, :, :], srcb)
            for t in range(ZR // ZEB):
                pltpu.sync_copy(zerob, acc_sh.at[pl.ds(sid * ZR + t * ZEB, ZEB)])
            plsc.subcore_barrier()
            _edge_pipeline(v_h, acc_sh, srcb, dstb, rows, semg, sems)
            plsc.subcore_barrier()
            for t in range(ZR // EB):
                sl = pl.ds(sid * ZR + t * EB, EB)
                pltpu.sync_copy(acc_sh.at[sl], out_h.at[ch, cid, sl])
            plsc.subcore_barrier()

    return k(vflat, srcoff, dst2d)


def _tc_deg_finalize(h0, h1, xp):
    def body(h0_ref, h1_ref, xp_ref, dis_ref, v1_ref):
        deg = 1.0 + h0_ref[...] + h1_ref[...]
        dis = lax.rsqrt(deg)
        dis_ref[...] = dis
        v1_ref[...] = dis * xp_ref[...]

    return pl.pallas_call(
        body,
        out_shape=(jax.ShapeDtypeStruct((NPAD, 16), jnp.float32),
                   jax.ShapeDtypeStruct((NPAD, 16), jnp.float32)),
    )(h0, h1, xp)


BLK = 512
MGRID = NPAD // BLK


def _tc_mlp_mid(s0, s1, v1, dis, w1p, b1, w2):
    def body(s0_ref, s1_ref, v1_ref, dis_ref, w1_ref, b1_ref, w2_ref, out_ref):
        z = dis_ref[...] * (s0_ref[...] + s1_ref[...] + v1_ref[...])
        h = jnp.dot(z, w1_ref[...], preferred_element_type=jnp.float32)
        h = jnp.maximum(h + b1_ref[...], 0.0)
        u = jnp.dot(h, w2_ref[...], preferred_element_type=jnp.float32)
        v = dis_ref[...][:, 0:1] * u
        for c in range(FCH):
            out_ref[c] = v[:, c * FW:(c + 1) * FW]

    return pl.pallas_call(
        body,
        grid=(MGRID,),
        in_specs=[
            pl.BlockSpec((BLK, 16), lambda i: (i, 0)),
            pl.BlockSpec((BLK, 16), lambda i: (i, 0)),
            pl.BlockSpec((BLK, 16), lambda i: (i, 0)),
            pl.BlockSpec((BLK, 16), lambda i: (i, 0)),
            pl.BlockSpec((16, 2048), lambda i: (0, 0)),
            pl.BlockSpec((1, 2048), lambda i: (0, 0)),
            pl.BlockSpec((2048, 1024), lambda i: (0, 0)),
        ],
        out_specs=pl.BlockSpec((FCH, BLK, FW), lambda i: (0, i, 0)),
        out_shape=jax.ShapeDtypeStruct((FCH, NPAD, FW), jnp.float32),
        compiler_params=pltpu.CompilerParams(
            dimension_semantics=("arbitrary",)),
    )(s0, s1, v1, dis, w1p, b1, w2)


def _tc_pool_head(a0, a1, v, dis, b2, batchp, wh1, bh1, wh2, bh2, wop, bop):
    def body(a0_ref, a1_ref, v_ref, dis_ref, b2_ref, bat_ref,
             wh1_ref, bh1_ref, wh2_ref, bh2_ref, wo_ref, bo_ref,
             out_ref, psum_ref, cnt_ref):
        i = pl.program_id(0)

        @pl.when(i == 0)
        def _():
            psum_ref[...] = jnp.zeros_like(psum_ref)
            cnt_ref[...] = jnp.zeros_like(cnt_ref)

        oh = (bat_ref[...] == lax.broadcasted_iota(jnp.int32, (1, 32), 1))
        oh = oh.astype(jnp.float32)
        dnums = (((0,), (0,)), ((), ()))
        cnt_ref[...] += lax.dot_general(
            oh, jnp.ones((BLK, FW), jnp.float32), dnums,
            preferred_element_type=jnp.float32)
        d = dis_ref[...][:, 0:1]
        for c in range(FCH):
            y = d * (a0_ref[c, 0] + a1_ref[c, 0] + v_ref[c])
            y = jnp.maximum(y + b2_ref[...][:, c * FW:(c + 1) * FW], 0.0)
            psum_ref[:, c * FW:(c + 1) * FW] += lax.dot_general(
                oh, y, dnums, preferred_element_type=jnp.float32)

        @pl.when(i == MGRID - 1)
        def _():
            g = psum_ref[...] / jnp.maximum(cnt_ref[...][:, 0:1], 1.0)
            g = jnp.dot(g, wh1_ref[...], preferred_element_type=jnp.float32)
            g = jnp.maximum(g + bh1_ref[...], 0.0)
            g = jnp.dot(g, wh2_ref[...], preferred_element_type=jnp.float32)
            g = jnp.maximum(g + bh2_ref[...], 0.0)
            out_ref[...] = (jnp.dot(g, wo_ref[...],
                                    preferred_element_type=jnp.float32)
                            + bo_ref[...])

    return pl.pallas_call(
        body,
        grid=(MGRID,),
        in_specs=[
            pl.BlockSpec((FCH, 1, BLK, FW), lambda i: (0, 0, i, 0)),
            pl.BlockSpec((FCH, 1, BLK, FW), lambda i: (0, 1, i, 0)),
            pl.BlockSpec((FCH, BLK, FW), lambda i: (0, i, 0)),
            pl.BlockSpec((BLK, 16), lambda i: (i, 0)),
            pl.BlockSpec((1, 1024), lambda i: (0, 0)),
            pl.BlockSpec((BLK, 1), lambda i: (i, 0)),
            pl.BlockSpec((1024, 1024), lambda i: (0, 0)),
            pl.BlockSpec((1, 1024), lambda i: (0, 0)),
            pl.BlockSpec((1024, 512), lambda i: (0, 0)),
            pl.BlockSpec((1, 512), lambda i: (0, 0)),
            pl.BlockSpec((512, 128), lambda i: (0, 0)),
            pl.BlockSpec((1, 128), lambda i: (0, 0)),
        ],
        out_specs=pl.BlockSpec((32, 128), lambda i: (0, 0)),
        out_shape=jax.ShapeDtypeStruct((32, 128), jnp.float32),
        scratch_shapes=[
            pltpu.VMEM((32, 1024), jnp.float32),
            pltpu.VMEM((32, 128), jnp.float32),
        ],
        compiler_params=pltpu.CompilerParams(
            dimension_semantics=("arbitrary",)),
    )(a0, a1, v, dis, b2, batchp, wh1, bh1, wh2, bh2, wop, bop)


def kernel(x, edge_index, batch, W1, b1, W2, b2, Wh1, bh1, Wh2, bh2, Wo, bo):
    f32, i32 = jnp.float32, jnp.int32

    xp = jnp.zeros((NPAD, 16), f32).at[:N_NODES, :5].set(x)
    npad_e = EPAD - E_EDGES
    ppw = npad_e // NW
    padidx = (DUMMY + jnp.arange(npad_e, dtype=i32) % (NPAD - N_NODES)
              ).reshape(NW, ppw)
    src2d = jnp.concatenate(
        [edge_index[0].reshape(NW, E_EDGES // NW), padidx], axis=1
    ).reshape(NW, NBATCH, EB)
    dst2d = jnp.concatenate(
        [edge_index[1].reshape(NW, E_EDGES // NW), padidx], axis=1
    ).reshape(NW, NBATCH, EB)
    srcoff = (src2d[None]
              + (jnp.arange(FCH, dtype=i32) * NPAD)[:, None, None, None])
    ones16 = jnp.ones((NPAD, 16), f32)
    batchp = jnp.concatenate(
        [batch, jnp.full((NPAD - N_NODES,), 99, i32)]).reshape(NPAD, 1)
    w1p = jnp.zeros((16, 2048), f32).at[:5].set(W1)
    wop = jnp.zeros((512, 128), f32).at[:, :2].set(Wo)
    bop = jnp.zeros((1, 128), f32).at[0, :2].set(bo)

    hist = _sc_scatter16(ones16, dst2d, dst2d)
    dis, v1 = _tc_deg_finalize(hist[0], hist[1], xp)
    s1 = _sc_scatter16(v1, src2d, dst2d)
    v = _tc_mlp_mid(s1[0], s1[1], v1, dis, w1p, b1.reshape(1, -1), W2)
    acc = _sc_scatter128(v.reshape(FCH * NPAD, FW), srcoff, dst2d)
    out = _tc_pool_head(acc, acc, v, dis, b2.reshape(1, -1),
                        batchp, Wh1, bh1.reshape(1, -1), Wh2,
                        bh2.reshape(1, -1), wop, bop)
    return out[:, :2]


# speedup vs baseline: 14.8986x; 1.0823x over previous
import functools

import jax
import jax.numpy as jnp
from jax import lax
from jax.experimental import pallas as pl
from jax.experimental.pallas import tpu as pltpu, tpu_sc as plsc

N_NODES = 10000
NPAD = 10240
E_EDGES = 160000
NC, NS = 2, 16
NW = NC * NS
EB = 128
EPAD = 163840
EPW = EPAD // NW
NBATCH = EPW // EB
ZR = NPAD // NS
FCH = 8
FW = 128
DUMMY = N_NODES

_mesh = plsc.VectorSubcoreMesh(
    core_axis_name="c", subcore_axis_name="s", num_cores=NC, num_subcores=NS)


def _zero_fill(buf, rows, width):
    vl = 32 if buf.dtype == jnp.bfloat16 else 16
    zv = jnp.zeros((vl,), buf.dtype)

    @pl.loop(0, rows)
    def _(r):
        for c in range(width // vl):
            buf[r, pl.ds(c * vl, vl)] = zv


NBUF = 4
LEAD = 2
ZEB = 32


def _edge_pipeline(table_h, acc_sh, srcb, dstb, rows, semg, sems):
    def wait_bytes(sem, slot):
        pltpu.make_async_copy(table_h.at[pl.ds(0, EB)], rows[slot], sem).wait()

    for b in range(LEAD):
        pltpu.async_copy(table_h.at[srcb.at[b]], rows[b], semg[b])

    @pl.loop(0, NBATCH // NBUF)
    def _(g):
        for b in range(NBUF):
            jb = g * NBUF + b
            wait_bytes(semg[b], b)
            pltpu.async_copy(rows[b], acc_sh.at[dstb.at[jb]], sems[b],
                             add=True)
            nslot = (b + LEAD) % NBUF
            nj = jb + LEAD

            @pl.when(nj < NBATCH)
            def _():
                @pl.when(nj >= NBUF)
                def _():
                    wait_bytes(sems[nslot], nslot)

                pltpu.async_copy(table_h.at[srcb.at[nj]], rows[nslot],
                                 semg[nslot])

    for b in range(NBUF):
        wait_bytes(sems[b], b)


def _sc_scatter16(table, src2d, dst2d):
    @functools.partial(
        pl.kernel,
        out_type=jax.ShapeDtypeStruct((NC, NPAD, 16), jnp.float32),
        mesh=_mesh,
        scratch_types=[
            pltpu.VMEM_SHARED((NPAD, 16), jnp.float32),
            pltpu.VMEM((NBATCH, EB), jnp.int32),
            pltpu.VMEM((NBATCH, EB), jnp.int32),
            [pltpu.VMEM((EB, 16), jnp.float32)] * NBUF,
            pltpu.VMEM((ZEB, 16), jnp.float32),
            [pltpu.SemaphoreType.DMA] * NBUF,
            [pltpu.SemaphoreType.DMA] * NBUF,
        ],
        compiler_params=pltpu.CompilerParams(use_tc_tiling_on_sc=False),
    )
    def k(table_h, src_h, dst_h, out_h, acc_sh, srcb, dstb, rows, zerob,
          semg, sems):
        cid = lax.axis_index("c")
        sid = lax.axis_index("s")
        wid = sid * NC + cid
        _zero_fill(zerob, ZEB, 16)
        pltpu.sync_copy(src_h.at[wid, :, :], srcb)
        pltpu.sync_copy(dst_h.at[wid, :, :], dstb)
        for t in range(ZR // ZEB):
            pltpu.sync_copy(zerob, acc_sh.at[pl.ds(sid * ZR + t * ZEB, ZEB)])
        plsc.subcore_barrier()
        _edge_pipeline(table_h, acc_sh, srcb, dstb, rows, semg, sems)
        plsc.subcore_barrier()
        for t in range(ZR // EB):
            sl = pl.ds(sid * ZR + t * EB, EB)
            pltpu.sync_copy(acc_sh.at[sl], out_h.at[cid, sl])

    return k(table, src2d, dst2d)


def _sc_scatter128(vflat, srcoff, dst2d):
    @functools.partial(
        pl.kernel,
        out_type=jax.ShapeDtypeStruct((FCH, NC, NPAD, FW), jnp.bfloat16),
        mesh=_mesh,
        scratch_types=[
            pltpu.VMEM_SHARED((NPAD, FW), jnp.bfloat16),
            pltpu.VMEM((NBATCH, EB), jnp.int32),
            pltpu.VMEM((NBATCH, EB), jnp.int32),
            [pltpu.VMEM((EB, FW), jnp.bfloat16)] * NBUF,
            pltpu.VMEM((ZEB, FW), jnp.bfloat16),
            [pltpu.SemaphoreType.DMA] * NBUF,
            [pltpu.SemaphoreType.DMA] * NBUF,
        ],
        compiler_params=pltpu.CompilerParams(use_tc_tiling_on_sc=False),
    )
    def k(v_h, srcoff_h, dst_h, out_h, acc_sh, srcb, dstb, rows, zerob,
          semg, sems):
        cid = lax.axis_index("c")
        sid = lax.axis_index("s")
        wid = sid * NC + cid
        _zero_fill(zerob, ZEB, FW)
        pltpu.sync_copy(dst_h.at[wid, :, :], dstb)

        @pl.loop(0, FCH)
        def _(ch):
            pltpu.sync_copy(srcoff_h.at[ch, wid, :, :], srcb)
            for t in range(ZR // ZEB):
                pltpu.sync_copy(zerob, acc_sh.at[pl.ds(sid * ZR + t * ZEB, ZEB)])
            plsc.subcore_barrier()
            _edge_pipeline(v_h, acc_sh, srcb, dstb, rows, semg, sems)
            plsc.subcore_barrier()
            for t in range(ZR // EB):
                sl = pl.ds(sid * ZR + t * EB, EB)
                pltpu.sync_copy(acc_sh.at[sl], out_h.at[ch, cid, sl])
            plsc.subcore_barrier()

    return k(vflat, srcoff, dst2d)


def _tc_deg_finalize(h0, h1, xp):
    def body(h0_ref, h1_ref, xp_ref, dis_ref, v1_ref):
        deg = 1.0 + h0_ref[...] + h1_ref[...]
        dis = lax.rsqrt(deg)
        dis_ref[...] = dis
        v1_ref[...] = dis * xp_ref[...]

    return pl.pallas_call(
        body,
        out_shape=(jax.ShapeDtypeStruct((NPAD, 16), jnp.float32),
                   jax.ShapeDtypeStruct((NPAD, 16), jnp.float32)),
    )(h0, h1, xp)


BLK = 512
MGRID = NPAD // BLK


def _tc_mlp_mid(s0, s1, v1, dis, w1p, b1, w2):
    def body(s0_ref, s1_ref, v1_ref, dis_ref, w1_ref, b1_ref, w2_ref, out_ref):
        z = dis_ref[...] * (s0_ref[...] + s1_ref[...] + v1_ref[...])
        h = jnp.dot(z, w1_ref[...], preferred_element_type=jnp.float32)
        h = jnp.maximum(h + b1_ref[...], 0.0)
        u = jnp.dot(h, w2_ref[...], preferred_element_type=jnp.float32)
        v = (dis_ref[...][:, 0:1] * u).astype(jnp.bfloat16)
        for c in range(FCH):
            out_ref[c] = v[:, c * FW:(c + 1) * FW]

    return pl.pallas_call(
        body,
        grid=(MGRID,),
        in_specs=[
            pl.BlockSpec((BLK, 16), lambda i: (i, 0)),
            pl.BlockSpec((BLK, 16), lambda i: (i, 0)),
            pl.BlockSpec((BLK, 16), lambda i: (i, 0)),
            pl.BlockSpec((BLK, 16), lambda i: (i, 0)),
            pl.BlockSpec((16, 2048), lambda i: (0, 0)),
            pl.BlockSpec((1, 2048), lambda i: (0, 0)),
            pl.BlockSpec((2048, 1024), lambda i: (0, 0)),
        ],
        out_specs=pl.BlockSpec((FCH, BLK, FW), lambda i: (0, i, 0)),
        out_shape=jax.ShapeDtypeStruct((FCH, NPAD, FW), jnp.bfloat16),
        compiler_params=pltpu.CompilerParams(
            dimension_semantics=("arbitrary",)),
    )(s0, s1, v1, dis, w1p, b1, w2)


def _tc_pool_head(a0, a1, v, dis, b2, batchp, wh1, bh1, wh2, bh2, wop, bop):
    def body(a0_ref, a1_ref, v_ref, dis_ref, b2_ref, bat_ref,
             wh1_ref, bh1_ref, wh2_ref, bh2_ref, wo_ref, bo_ref,
             out_ref, psum_ref, cnt_ref):
        i = pl.program_id(0)

        @pl.when(i == 0)
        def _():
            psum_ref[...] = jnp.zeros_like(psum_ref)
            cnt_ref[...] = jnp.zeros_like(cnt_ref)

        oh = (bat_ref[...] == lax.broadcasted_iota(jnp.int32, (1, 32), 1))
        oh = oh.astype(jnp.float32)
        dnums = (((0,), (0,)), ((), ()))
        cnt_ref[...] += lax.dot_general(
            oh, jnp.ones((BLK, FW), jnp.float32), dnums,
            preferred_element_type=jnp.float32)
        d = dis_ref[...][:, 0:1]
        for c in range(FCH):
            y = d * (a0_ref[c, 0].astype(jnp.float32)
                     + a1_ref[c, 0].astype(jnp.float32)
                     + v_ref[c].astype(jnp.float32))
            y = jnp.maximum(y + b2_ref[...][:, c * FW:(c + 1) * FW], 0.0)
            psum_ref[:, c * FW:(c + 1) * FW] += lax.dot_general(
                oh, y, dnums, preferred_element_type=jnp.float32)

        @pl.when(i == MGRID - 1)
        def _():
            g = psum_ref[...] / jnp.maximum(cnt_ref[...][:, 0:1], 1.0)
            g = jnp.dot(g, wh1_ref[...], preferred_element_type=jnp.float32)
            g = jnp.maximum(g + bh1_ref[...], 0.0)
            g = jnp.dot(g, wh2_ref[...], preferred_element_type=jnp.float32)
            g = jnp.maximum(g + bh2_ref[...], 0.0)
            out_ref[...] = (jnp.dot(g, wo_ref[...],
                                    preferred_element_type=jnp.float32)
                            + bo_ref[...])

    return pl.pallas_call(
        body,
        grid=(MGRID,),
        in_specs=[
            pl.BlockSpec((FCH, 1, BLK, FW), lambda i: (0, 0, i, 0)),
            pl.BlockSpec((FCH, 1, BLK, FW), lambda i: (0, 1, i, 0)),
            pl.BlockSpec((FCH, BLK, FW), lambda i: (0, i, 0)),
            pl.BlockSpec((BLK, 16), lambda i: (i, 0)),
            pl.BlockSpec((1, 1024), lambda i: (0, 0)),
            pl.BlockSpec((BLK, 1), lambda i: (i, 0)),
            pl.BlockSpec((1024, 1024), lambda i: (0, 0)),
            pl.BlockSpec((1, 1024), lambda i: (0, 0)),
            pl.BlockSpec((1024, 512), lambda i: (0, 0)),
            pl.BlockSpec((1, 512), lambda i: (0, 0)),
            pl.BlockSpec((512, 128), lambda i: (0, 0)),
            pl.BlockSpec((1, 128), lambda i: (0, 0)),
        ],
        out_specs=pl.BlockSpec((32, 128), lambda i: (0, 0)),
        out_shape=jax.ShapeDtypeStruct((32, 128), jnp.float32),
        scratch_shapes=[
            pltpu.VMEM((32, 1024), jnp.float32),
            pltpu.VMEM((32, 128), jnp.float32),
        ],
        compiler_params=pltpu.CompilerParams(
            dimension_semantics=("arbitrary",)),
    )(a0, a1, v, dis, b2, batchp, wh1, bh1, wh2, bh2, wop, bop)


def kernel(x, edge_index, batch, W1, b1, W2, b2, Wh1, bh1, Wh2, bh2, Wo, bo):
    f32, i32 = jnp.float32, jnp.int32

    xp = jnp.zeros((NPAD, 16), f32).at[:N_NODES, :5].set(x)
    npad_e = EPAD - E_EDGES
    ppw = npad_e // NW
    padidx = (DUMMY + jnp.arange(npad_e, dtype=i32) % (NPAD - N_NODES)
              ).reshape(NW, ppw)
    src2d = jnp.concatenate(
        [edge_index[0].reshape(NW, E_EDGES // NW), padidx], axis=1
    ).reshape(NW, NBATCH, EB)
    dst2d = jnp.concatenate(
        [edge_index[1].reshape(NW, E_EDGES // NW), padidx], axis=1
    ).reshape(NW, NBATCH, EB)
    srcoff = (src2d[None]
              + (jnp.arange(FCH, dtype=i32) * NPAD)[:, None, None, None])
    ones16 = jnp.ones((NPAD, 16), f32)
    batchp = jnp.concatenate(
        [batch, jnp.full((NPAD - N_NODES,), 99, i32)]).reshape(NPAD, 1)
    w1p = jnp.zeros((16, 2048), f32).at[:5].set(W1)
    wop = jnp.zeros((512, 128), f32).at[:, :2].set(Wo)
    bop = jnp.zeros((1, 128), f32).at[0, :2].set(bo)

    hist = _sc_scatter16(ones16, dst2d, dst2d)
    dis, v1 = _tc_deg_finalize(hist[0], hist[1], xp)
    s1 = _sc_scatter16(v1, src2d, dst2d)
    v = _tc_mlp_mid(s1[0], s1[1], v1, dis, w1p, b1.reshape(1, -1), W2)
    acc = _sc_scatter128(v.reshape(FCH * NPAD, FW), srcoff, dst2d)
    out = _tc_pool_head(acc, acc, v, dis, b2.reshape(1, -1),
                        batchp, Wh1, bh1.reshape(1, -1), Wh2,
                        bh2.reshape(1, -1), wop, bop)
    return out[:, :2]


# speedup vs baseline: 15.1314x; 1.0156x over previous
import functools

import jax
import jax.numpy as jnp
from jax import lax
from jax.experimental import pallas as pl
from jax.experimental.pallas import tpu as pltpu, tpu_sc as plsc

N_NODES = 10000
NPAD = 10240
E_EDGES = 160000
NC, NS = 2, 16
NW = NC * NS
EB = 128
EPAD = 163840
EPW = EPAD // NW
NBATCH = EPW // EB
ZR = NPAD // NS
FCH = 8
FW = 128
DUMMY = N_NODES

_mesh = plsc.VectorSubcoreMesh(
    core_axis_name="c", subcore_axis_name="s", num_cores=NC, num_subcores=NS)


def _zero_fill(buf, rows, width):
    vl = 32 if buf.dtype == jnp.bfloat16 else 16
    zv = jnp.zeros((vl,), buf.dtype)

    @pl.loop(0, rows)
    def _(r):
        for c in range(width // vl):
            buf[r, pl.ds(c * vl, vl)] = zv


NBUF = 4
LEAD = 2
ZEB = 32


def _edge_pipeline(table_h, acc_sh, srcb, dstb, rows, semg, sems):
    def wait_bytes(sem, slot):
        pltpu.make_async_copy(table_h.at[pl.ds(0, EB)], rows[slot], sem).wait()

    for b in range(LEAD):
        pltpu.async_copy(table_h.at[srcb.at[b]], rows[b], semg[b])

    @pl.loop(0, NBATCH // NBUF)
    def _(g):
        for b in range(NBUF):
            jb = g * NBUF + b
            wait_bytes(semg[b], b)
            pltpu.async_copy(rows[b], acc_sh.at[dstb.at[jb]], sems[b],
                             add=True)
            nslot = (b + LEAD) % NBUF
            nj = jb + LEAD

            @pl.when(nj < NBATCH)
            def _():
                @pl.when(nj >= NBUF)
                def _():
                    wait_bytes(sems[nslot], nslot)

                pltpu.async_copy(table_h.at[srcb.at[nj]], rows[nslot],
                                 semg[nslot])

    for b in range(NBUF):
        wait_bytes(sems[b], b)


def _sc_scatter16(table, src2d, dst2d):
    @functools.partial(
        pl.kernel,
        out_type=jax.ShapeDtypeStruct((NC, NPAD, 16), jnp.float32),
        mesh=_mesh,
        scratch_types=[
            pltpu.VMEM_SHARED((NPAD, 16), jnp.float32),
            pltpu.VMEM((NBATCH, EB), jnp.int32),
            pltpu.VMEM((NBATCH, EB), jnp.int32),
            [pltpu.VMEM((EB, 16), jnp.float32)] * NBUF,
            pltpu.VMEM((ZEB, 16), jnp.float32),
            [pltpu.SemaphoreType.DMA] * NBUF,
            [pltpu.SemaphoreType.DMA] * NBUF,
        ],
        compiler_params=pltpu.CompilerParams(use_tc_tiling_on_sc=False),
    )
    def k(table_h, src_h, dst_h, out_h, acc_sh, srcb, dstb, rows, zerob,
          semg, sems):
        cid = lax.axis_index("c")
        sid = lax.axis_index("s")
        wid = sid * NC + cid
        _zero_fill(zerob, ZEB, 16)
        pltpu.sync_copy(src_h.at[wid, :, :], srcb)
        pltpu.sync_copy(dst_h.at[wid, :, :], dstb)
        for t in range(ZR // ZEB):
            pltpu.sync_copy(zerob, acc_sh.at[pl.ds(sid * ZR + t * ZEB, ZEB)])
        plsc.subcore_barrier()
        _edge_pipeline(table_h, acc_sh, srcb, dstb, rows, semg, sems)
        plsc.subcore_barrier()
        for t in range(ZR // EB):
            sl = pl.ds(sid * ZR + t * EB, EB)
            pltpu.sync_copy(acc_sh.at[sl], out_h.at[cid, sl])

    return k(table, src2d, dst2d)


def _sc_scatter128(vflat, srcoff, dst2d):
    nch = srcoff.shape[0]

    @functools.partial(
        pl.kernel,
        out_type=jax.ShapeDtypeStruct((nch, NC, NPAD, FW), jnp.bfloat16),
        mesh=_mesh,
        scratch_types=[
            pltpu.VMEM_SHARED((NPAD, FW), jnp.bfloat16),
            pltpu.VMEM((NBATCH, EB), jnp.int32),
            pltpu.VMEM((NBATCH, EB), jnp.int32),
            [pltpu.VMEM((EB, FW), jnp.bfloat16)] * NBUF,
            pltpu.VMEM((ZEB, FW), jnp.bfloat16),
            [pltpu.SemaphoreType.DMA] * NBUF,
            [pltpu.SemaphoreType.DMA] * NBUF,
        ],
        compiler_params=pltpu.CompilerParams(use_tc_tiling_on_sc=False),
    )
    def k(v_h, srcoff_h, dst_h, out_h, acc_sh, srcb, dstb, rows, zerob,
          semg, sems):
        cid = lax.axis_index("c")
        sid = lax.axis_index("s")
        wid = sid * NC + cid
        _zero_fill(zerob, ZEB, FW)
        pltpu.sync_copy(dst_h.at[wid, :, :], dstb)

        @pl.loop(0, nch)
        def _(ch):
            pltpu.sync_copy(srcoff_h.at[ch, wid, :, :], srcb)
            for t in range(ZR // ZEB):
                pltpu.sync_copy(zerob, acc_sh.at[pl.ds(sid * ZR + t * ZEB, ZEB)])
            plsc.subcore_barrier()
            _edge_pipeline(v_h, acc_sh, srcb, dstb, rows, semg, sems)
            plsc.subcore_barrier()
            for t in range(ZR // EB):
                sl = pl.ds(sid * ZR + t * EB, EB)
                pltpu.sync_copy(acc_sh.at[sl], out_h.at[ch, cid, sl])
            plsc.subcore_barrier()

    return k(vflat, srcoff, dst2d)


def _tc_deg_finalize(h0, h1, xp):
    def body(h0_ref, h1_ref, xp_ref, dis_ref, v1_ref):
        deg = 1.0 + h0_ref[...] + h1_ref[...]
        dis = lax.rsqrt(deg)
        dis_ref[...] = dis
        v1_ref[...] = dis * xp_ref[...]

    return pl.pallas_call(
        body,
        out_shape=(jax.ShapeDtypeStruct((NPAD, 16), jnp.float32),
                   jax.ShapeDtypeStruct((NPAD, 16), jnp.float32)),
    )(h0, h1, xp)


BLK = 512
MGRID = NPAD // BLK


def _tc_mlp_mid(s0, s1, v1, dis, w1p, b1, w2):
    def body(s0_ref, s1_ref, v1_ref, dis_ref, w1_ref, b1_ref, w2_ref, out_ref):
        z = dis_ref[...] * (s0_ref[...] + s1_ref[...] + v1_ref[...])
        h = jnp.dot(z, w1_ref[...], preferred_element_type=jnp.float32)
        h = jnp.maximum(h + b1_ref[...], 0.0)
        u = jnp.dot(h, w2_ref[...], preferred_element_type=jnp.float32)
        v = (dis_ref[...][:, 0:1] * u).astype(jnp.bfloat16)
        for c in range(FCH):
            out_ref[c] = v[:, c * FW:(c + 1) * FW]

    return pl.pallas_call(
        body,
        grid=(MGRID,),
        in_specs=[
            pl.BlockSpec((BLK, 16), lambda i: (i, 0)),
            pl.BlockSpec((BLK, 16), lambda i: (i, 0)),
            pl.BlockSpec((BLK, 16), lambda i: (i, 0)),
            pl.BlockSpec((BLK, 16), lambda i: (i, 0)),
            pl.BlockSpec((16, 2048), lambda i: (0, 0)),
            pl.BlockSpec((1, 2048), lambda i: (0, 0)),
            pl.BlockSpec((2048, 1024), lambda i: (0, 0)),
        ],
        out_specs=pl.BlockSpec((FCH, BLK, FW), lambda i: (0, i, 0)),
        out_shape=jax.ShapeDtypeStruct((FCH, NPAD, FW), jnp.bfloat16),
        compiler_params=pltpu.CompilerParams(
            dimension_semantics=("arbitrary",)),
    )(s0, s1, v1, dis, w1p, b1, w2)


def _tc_pool_half(a0, a1, v, dis, b2, batchp):
    hch = v.shape[0]
    hw = hch * FW

    def body(a0_ref, a1_ref, v_ref, dis_ref, b2_ref, bat_ref,
             out_ref, psum_ref):
        i = pl.program_id(0)

        @pl.when(i == 0)
        def _():
            psum_ref[...] = jnp.zeros_like(psum_ref)

        oh = (bat_ref[...] == lax.broadcasted_iota(jnp.int32, (1, 32), 1))
        oh = oh.astype(jnp.float32)
        dnums = (((0,), (0,)), ((), ()))
        d = dis_ref[...][:, 0:1]
        for c in range(hch):
            y = d * (a0_ref[c, 0].astype(jnp.float32)
                     + a1_ref[c, 0].astype(jnp.float32)
                     + v_ref[c].astype(jnp.float32))
            y = jnp.maximum(y + b2_ref[...][:, c * FW:(c + 1) * FW], 0.0)
            psum_ref[:, c * FW:(c + 1) * FW] += lax.dot_general(
                oh, y, dnums, preferred_element_type=jnp.float32)

        @pl.when(i == MGRID - 1)
        def _():
            out_ref[...] = psum_ref[...]

    return pl.pallas_call(
        body,
        grid=(MGRID,),
        in_specs=[
            pl.BlockSpec((hch, 1, BLK, FW), lambda i: (0, 0, i, 0)),
            pl.BlockSpec((hch, 1, BLK, FW), lambda i: (0, 1, i, 0)),
            pl.BlockSpec((hch, BLK, FW), lambda i: (0, i, 0)),
            pl.BlockSpec((BLK, 16), lambda i: (i, 0)),
            pl.BlockSpec((1, hw), lambda i: (0, 0)),
            pl.BlockSpec((BLK, 1), lambda i: (i, 0)),
        ],
        out_specs=pl.BlockSpec((32, hw), lambda i: (0, 0)),
        out_shape=jax.ShapeDtypeStruct((32, hw), jnp.float32),
        scratch_shapes=[
            pltpu.VMEM((32, hw), jnp.float32),
        ],
        compiler_params=pltpu.CompilerParams(
            dimension_semantics=("arbitrary",)),
    )(a0, a1, v, dis, b2, batchp)


def _tc_head(p0, p1, batchp, wh1, bh1, wh2, bh2, wop, bop):
    def body(p0_ref, p1_ref, bat_ref, wh1_ref, bh1_ref, wh2_ref, bh2_ref,
             wo_ref, bo_ref, out_ref):
        oh = (bat_ref[...] == lax.broadcasted_iota(jnp.int32, (1, 32), 1))
        oh = oh.astype(jnp.float32)
        dnums = (((0,), (0,)), ((), ()))
        cnt = lax.dot_general(oh, jnp.ones((NPAD, 128), jnp.float32), dnums,
                              preferred_element_type=jnp.float32)
        g = jnp.concatenate([p0_ref[...], p1_ref[...]], axis=1)
        g = g / jnp.maximum(cnt[:, 0:1], 1.0)
        g = jnp.dot(g, wh1_ref[...], preferred_element_type=jnp.float32)
        g = jnp.maximum(g + bh1_ref[...], 0.0)
        g = jnp.dot(g, wh2_ref[...], preferred_element_type=jnp.float32)
        g = jnp.maximum(g + bh2_ref[...], 0.0)
        out_ref[...] = (jnp.dot(g, wo_ref[...],
                                preferred_element_type=jnp.float32)
                        + bo_ref[...])

    return pl.pallas_call(
        body,
        out_shape=jax.ShapeDtypeStruct((32, 128), jnp.float32),
    )(p0, p1, batchp, wh1, bh1, wh2, bh2, wop, bop)


def kernel(x, edge_index, batch, W1, b1, W2, b2, Wh1, bh1, Wh2, bh2, Wo, bo):
    f32, i32 = jnp.float32, jnp.int32

    xp = jnp.zeros((NPAD, 16), f32).at[:N_NODES, :5].set(x)
    npad_e = EPAD - E_EDGES
    ppw = npad_e // NW
    padidx = (DUMMY + jnp.arange(npad_e, dtype=i32) % (NPAD - N_NODES)
              ).reshape(NW, ppw)
    src2d = jnp.concatenate(
        [edge_index[0].reshape(NW, E_EDGES // NW), padidx], axis=1
    ).reshape(NW, NBATCH, EB)
    dst2d = jnp.concatenate(
        [edge_index[1].reshape(NW, E_EDGES // NW), padidx], axis=1
    ).reshape(NW, NBATCH, EB)
    srcoff4 = (src2d[None]
               + (jnp.arange(FCH // 2, dtype=i32) * NPAD)[:, None, None,
                                                          None])
    ones16 = jnp.ones((NPAD, 16), f32)
    batchp = jnp.concatenate(
        [batch, jnp.full((NPAD - N_NODES,), 99, i32)]).reshape(NPAD, 1)
    w1p = jnp.zeros((16, 2048), f32).at[:5].set(W1)
    wop = jnp.zeros((512, 128), f32).at[:, :2].set(Wo)
    bop = jnp.zeros((1, 128), f32).at[0, :2].set(bo)

    hist = _sc_scatter16(ones16, dst2d, dst2d)
    dis, v1 = _tc_deg_finalize(hist[0], hist[1], xp)
    s1 = _sc_scatter16(v1, src2d, dst2d)
    v = _tc_mlp_mid(s1[0], s1[1], v1, dis, w1p, b1.reshape(1, -1), W2)
    vflat = v.reshape(FCH * NPAD, FW)
    h4 = FCH // 2 * NPAD
    acc_a = _sc_scatter128(vflat[:h4], srcoff4, dst2d)
    acc_b = _sc_scatter128(vflat[h4:], srcoff4, dst2d)
    b2r = b2.reshape(1, -1)
    p0 = _tc_pool_half(acc_a, acc_a, v[:FCH // 2], dis, b2r[:, :512], batchp)
    p1 = _tc_pool_half(acc_b, acc_b, v[FCH // 2:], dis, b2r[:, 512:], batchp)
    out = _tc_head(p0, p1, batchp, Wh1, bh1.reshape(1, -1), Wh2,
                   bh2.reshape(1, -1), wop, bop)
    return out[:, :2]


# speedup vs baseline: 15.2556x; 1.0082x over previous
import functools

import jax
import jax.numpy as jnp
from jax import lax
from jax.experimental import pallas as pl
from jax.experimental.pallas import tpu as pltpu, tpu_sc as plsc

N_NODES = 10000
NPAD = 10240
E_EDGES = 160000
NC, NS = 2, 16
NW = NC * NS
EB = 128
EPAD = 163840
EPW = EPAD // NW
NBATCH = EPW // EB
ZR = NPAD // NS
FCH = 8
FW = 128
DUMMY = N_NODES

_mesh = plsc.VectorSubcoreMesh(
    core_axis_name="c", subcore_axis_name="s", num_cores=NC, num_subcores=NS)


def _zero_fill(buf, rows, width):
    vl = 32 if buf.dtype == jnp.bfloat16 else 16
    zv = jnp.zeros((vl,), buf.dtype)

    @pl.loop(0, rows)
    def _(r):
        for c in range(width // vl):
            buf[r, pl.ds(c * vl, vl)] = zv


NBUF = 4
LEAD = 2
ZEB = 32


def _edge_pipeline(table_h, acc_sh, srcb, dstb, rows, semg, sems):
    def wait_bytes(sem, slot):
        pltpu.make_async_copy(table_h.at[pl.ds(0, EB)], rows[slot], sem).wait()

    for b in range(LEAD):
        pltpu.async_copy(table_h.at[srcb.at[b]], rows[b], semg[b])

    @pl.loop(0, NBATCH // NBUF)
    def _(g):
        for b in range(NBUF):
            jb = g * NBUF + b
            wait_bytes(semg[b], b)
            pltpu.async_copy(rows[b], acc_sh.at[dstb.at[jb]], sems[b],
                             add=True)
            nslot = (b + LEAD) % NBUF
            nj = jb + LEAD

            @pl.when(nj < NBATCH)
            def _():
                @pl.when(nj >= NBUF)
                def _():
                    wait_bytes(sems[nslot], nslot)

                pltpu.async_copy(table_h.at[srcb.at[nj]], rows[nslot],
                                 semg[nslot])

    for b in range(NBUF):
        wait_bytes(sems[b], b)


def _sc_scatter16(table, src2d, dst2d):
    @functools.partial(
        pl.kernel,
        out_type=jax.ShapeDtypeStruct((NC, NPAD, 16), jnp.float32),
        mesh=_mesh,
        scratch_types=[
            pltpu.VMEM_SHARED((NPAD, 16), jnp.float32),
            pltpu.VMEM((NBATCH, EB), jnp.int32),
            pltpu.VMEM((NBATCH, EB), jnp.int32),
            [pltpu.VMEM((EB, 16), jnp.float32)] * NBUF,
            pltpu.VMEM((ZR, 16), jnp.float32),
            [pltpu.SemaphoreType.DMA] * NBUF,
            [pltpu.SemaphoreType.DMA] * NBUF,
        ],
        compiler_params=pltpu.CompilerParams(use_tc_tiling_on_sc=False),
    )
    def k(table_h, src_h, dst_h, out_h, acc_sh, srcb, dstb, rows, zerob,
          semg, sems):
        cid = lax.axis_index("c")
        sid = lax.axis_index("s")
        wid = sid * NC + cid
        _zero_fill(zerob, ZR, 16)
        pltpu.sync_copy(src_h.at[wid, :, :], srcb)
        pltpu.sync_copy(dst_h.at[wid, :, :], dstb)
        sl = pl.ds(sid * ZR, ZR)
        pltpu.sync_copy(zerob, acc_sh.at[sl])
        plsc.subcore_barrier()
        _edge_pipeline(table_h, acc_sh, srcb, dstb, rows, semg, sems)
        plsc.subcore_barrier()
        pltpu.sync_copy(acc_sh.at[sl], out_h.at[cid, sl])

    return k(table, src2d, dst2d)


def _sc_scatter128(vflat, srcoff, dst2d):
    nch = srcoff.shape[0]

    @functools.partial(
        pl.kernel,
        out_type=jax.ShapeDtypeStruct((nch, NC, NPAD, FW), jnp.bfloat16),
        mesh=_mesh,
        scratch_types=[
            pltpu.VMEM_SHARED((NPAD, FW), jnp.bfloat16),
            pltpu.VMEM((NBATCH, EB), jnp.int32),
            pltpu.VMEM((NBATCH, EB), jnp.int32),
            [pltpu.VMEM((EB, FW), jnp.bfloat16)] * NBUF,
            pltpu.VMEM((ZR, FW), jnp.bfloat16),
            [pltpu.SemaphoreType.DMA] * NBUF,
            [pltpu.SemaphoreType.DMA] * NBUF,
        ],
        compiler_params=pltpu.CompilerParams(use_tc_tiling_on_sc=False),
    )
    def k(v_h, srcoff_h, dst_h, out_h, acc_sh, srcb, dstb, rows, zerob,
          semg, sems):
        cid = lax.axis_index("c")
        sid = lax.axis_index("s")
        wid = sid * NC + cid
        _zero_fill(zerob, ZR, FW)
        pltpu.sync_copy(dst_h.at[wid, :, :], dstb)
        sl = pl.ds(sid * ZR, ZR)

        @pl.loop(0, nch)
        def _(ch):
            pltpu.sync_copy(srcoff_h.at[ch, wid, :, :], srcb)
            pltpu.sync_copy(zerob, acc_sh.at[sl])
            plsc.subcore_barrier()
            _edge_pipeline(v_h, acc_sh, srcb, dstb, rows, semg, sems)
            plsc.subcore_barrier()
            pltpu.sync_copy(acc_sh.at[sl], out_h.at[ch, cid, sl])
            plsc.subcore_barrier()

    return k(vflat, srcoff, dst2d)


def _tc_deg_finalize(h0, h1, xp):
    def body(h0_ref, h1_ref, xp_ref, dis_ref, v1_ref):
        deg = 1.0 + h0_ref[...] + h1_ref[...]
        dis = lax.rsqrt(deg)
        dis_ref[...] = dis
        v1_ref[...] = dis * xp_ref[...]

    return pl.pallas_call(
        body,
        out_shape=(jax.ShapeDtypeStruct((NPAD, 16), jnp.float32),
                   jax.ShapeDtypeStruct((NPAD, 16), jnp.float32)),
    )(h0, h1, xp)


BLK = 512
MGRID = NPAD // BLK


def _tc_mlp_mid(s0, s1, v1, dis, w1p, b1, w2):
    def body(s0_ref, s1_ref, v1_ref, dis_ref, w1_ref, b1_ref, w2_ref, out_ref):
        z = dis_ref[...] * (s0_ref[...] + s1_ref[...] + v1_ref[...])
        h = jnp.dot(z, w1_ref[...], preferred_element_type=jnp.float32)
        h = jnp.maximum(h + b1_ref[...], 0.0)
        u = jnp.dot(h, w2_ref[...], preferred_element_type=jnp.float32)
        v = (dis_ref[...][:, 0:1] * u).astype(jnp.bfloat16)
        for c in range(FCH):
            out_ref[c] = v[:, c * FW:(c + 1) * FW]

    return pl.pallas_call(
        body,
        grid=(MGRID,),
        in_specs=[
            pl.BlockSpec((BLK, 16), lambda i: (i, 0)),
            pl.BlockSpec((BLK, 16), lambda i: (i, 0)),
            pl.BlockSpec((BLK, 16), lambda i: (i, 0)),
            pl.BlockSpec((BLK, 16), lambda i: (i, 0)),
            pl.BlockSpec((16, 2048), lambda i: (0, 0)),
            pl.BlockSpec((1, 2048), lambda i: (0, 0)),
            pl.BlockSpec((2048, 1024), lambda i: (0, 0)),
        ],
        out_specs=pl.BlockSpec((FCH, BLK, FW), lambda i: (0, i, 0)),
        out_shape=jax.ShapeDtypeStruct((FCH, NPAD, FW), jnp.bfloat16),
        compiler_params=pltpu.CompilerParams(
            dimension_semantics=("arbitrary",)),
    )(s0, s1, v1, dis, w1p, b1, w2)


def _tc_pool_half(a0, a1, v, dis, b2, batchp):
    hch = v.shape[0]
    hw = hch * FW

    def body(a0_ref, a1_ref, v_ref, dis_ref, b2_ref, bat_ref,
             out_ref, psum_ref):
        i = pl.program_id(0)

        @pl.when(i == 0)
        def _():
            psum_ref[...] = jnp.zeros_like(psum_ref)

        oh = (bat_ref[...] == lax.broadcasted_iota(jnp.int32, (1, 32), 1))
        oh = oh.astype(jnp.float32)
        dnums = (((0,), (0,)), ((), ()))
        d = dis_ref[...][:, 0:1]
        for c in range(hch):
            y = d * (a0_ref[c, 0].astype(jnp.float32)
                     + a1_ref[c, 0].astype(jnp.float32)
                     + v_ref[c].astype(jnp.float32))
            y = jnp.maximum(y + b2_ref[...][:, c * FW:(c + 1) * FW], 0.0)
            psum_ref[:, c * FW:(c + 1) * FW] += lax.dot_general(
                oh, y, dnums, preferred_element_type=jnp.float32)

        @pl.when(i == MGRID - 1)
        def _():
            out_ref[...] = psum_ref[...]

    return pl.pallas_call(
        body,
        grid=(MGRID,),
        in_specs=[
            pl.BlockSpec((hch, 1, BLK, FW), lambda i: (0, 0, i, 0)),
            pl.BlockSpec((hch, 1, BLK, FW), lambda i: (0, 1, i, 0)),
            pl.BlockSpec((hch, BLK, FW), lambda i: (0, i, 0)),
            pl.BlockSpec((BLK, 16), lambda i: (i, 0)),
            pl.BlockSpec((1, hw), lambda i: (0, 0)),
            pl.BlockSpec((BLK, 1), lambda i: (i, 0)),
        ],
        out_specs=pl.BlockSpec((32, hw), lambda i: (0, 0)),
        out_shape=jax.ShapeDtypeStruct((32, hw), jnp.float32),
        scratch_shapes=[
            pltpu.VMEM((32, hw), jnp.float32),
        ],
        compiler_params=pltpu.CompilerParams(
            dimension_semantics=("arbitrary",)),
    )(a0, a1, v, dis, b2, batchp)


def _tc_head(p0, p1, batchp, wh1, bh1, wh2, bh2, wop, bop):
    def body(p0_ref, p1_ref, bat_ref, wh1_ref, bh1_ref, wh2_ref, bh2_ref,
             wo_ref, bo_ref, out_ref):
        oh = (bat_ref[...] == lax.broadcasted_iota(jnp.int32, (1, 32), 1))
        oh = oh.astype(jnp.float32)
        dnums = (((0,), (0,)), ((), ()))
        cnt = lax.dot_general(oh, jnp.ones((NPAD, 128), jnp.float32), dnums,
                              preferred_element_type=jnp.float32)
        g = jnp.concatenate([p0_ref[...], p1_ref[...]], axis=1)
        g = g / jnp.maximum(cnt[:, 0:1], 1.0)
        g = jnp.dot(g, wh1_ref[...], preferred_element_type=jnp.float32)
        g = jnp.maximum(g + bh1_ref[...], 0.0)
        g = jnp.dot(g, wh2_ref[...], preferred_element_type=jnp.float32)
        g = jnp.maximum(g + bh2_ref[...], 0.0)
        out_ref[...] = (jnp.dot(g, wo_ref[...],
                                preferred_element_type=jnp.float32)
                        + bo_ref[...])

    return pl.pallas_call(
        body,
        out_shape=jax.ShapeDtypeStruct((32, 128), jnp.float32),
    )(p0, p1, batchp, wh1, bh1, wh2, bh2, wop, bop)


def kernel(x, edge_index, batch, W1, b1, W2, b2, Wh1, bh1, Wh2, bh2, Wo, bo):
    f32, i32 = jnp.float32, jnp.int32

    xp = jnp.zeros((NPAD, 16), f32).at[:N_NODES, :5].set(x)
    npad_e = EPAD - E_EDGES
    ppw = npad_e // NW
    padidx = (DUMMY + jnp.arange(npad_e, dtype=i32) % (NPAD - N_NODES)
              ).reshape(NW, ppw)
    src2d = jnp.concatenate(
        [edge_index[0].reshape(NW, E_EDGES // NW), padidx], axis=1
    ).reshape(NW, NBATCH, EB)
    dst2d = jnp.concatenate(
        [edge_index[1].reshape(NW, E_EDGES // NW), padidx], axis=1
    ).reshape(NW, NBATCH, EB)
    srcoff4 = (src2d[None]
               + (jnp.arange(FCH // 2, dtype=i32) * NPAD)[:, None, None,
                                                          None])
    ones16 = jnp.ones((NPAD, 16), f32)
    batchp = jnp.concatenate(
        [batch, jnp.full((NPAD - N_NODES,), 99, i32)]).reshape(NPAD, 1)
    w1p = jnp.zeros((16, 2048), f32).at[:5].set(W1)
    wop = jnp.zeros((512, 128), f32).at[:, :2].set(Wo)
    bop = jnp.zeros((1, 128), f32).at[0, :2].set(bo)

    hist = _sc_scatter16(ones16, dst2d, dst2d)
    dis, v1 = _tc_deg_finalize(hist[0], hist[1], xp)
    s1 = _sc_scatter16(v1, src2d, dst2d)
    v = _tc_mlp_mid(s1[0], s1[1], v1, dis, w1p, b1.reshape(1, -1), W2)
    vflat = v.reshape(FCH * NPAD, FW)
    h4 = FCH // 2 * NPAD
    acc_a = _sc_scatter128(vflat[:h4], srcoff4, dst2d)
    acc_b = _sc_scatter128(vflat[h4:], srcoff4, dst2d)
    b2r = b2.reshape(1, -1)
    p0 = _tc_pool_half(acc_a, acc_a, v[:FCH // 2], dis, b2r[:, :512], batchp)
    p1 = _tc_pool_half(acc_b, acc_b, v[FCH // 2:], dis, b2r[:, 512:], batchp)
    out = _tc_head(p0, p1, batchp, Wh1, bh1.reshape(1, -1), Wh2,
                   bh2.reshape(1, -1), wop, bop)
    return out[:, :2]


# speedup vs baseline: 15.7411x; 1.0318x over previous
import functools

import jax
import jax.numpy as jnp
from jax import lax
from jax.experimental import pallas as pl
from jax.experimental.pallas import tpu as pltpu, tpu_sc as plsc

N_NODES = 10000
NPAD = 10240
E_EDGES = 160000
NC, NS = 2, 16
NW = NC * NS
EB = 128
EPAD = 163840
EPW = EPAD // NW
NBATCH = EPW // EB
ZR = NPAD // NS
FCH = 8
FW = 128
DUMMY = N_NODES

_mesh = plsc.VectorSubcoreMesh(
    core_axis_name="c", subcore_axis_name="s", num_cores=NC, num_subcores=NS)


def _zero_fill(buf, rows, width):
    vl = 32 if buf.dtype == jnp.bfloat16 else 16
    zv = jnp.zeros((vl,), buf.dtype)

    @pl.loop(0, rows)
    def _(r):
        for c in range(width // vl):
            buf[r, pl.ds(c * vl, vl)] = zv


NBUF = 4
LEAD = 2
ZEB = 32


def _edge_pipeline(table_h, dummy_h, acc_sh, srcb, dstb, rows, semg, sems):
    def wait_bytes(sem, slot):
        pltpu.make_async_copy(dummy_h, rows[slot], sem).wait()

    for b in range(LEAD):
        pltpu.async_copy(table_h.at[srcb.at[b]], rows[b], semg[b])

    @pl.loop(0, NBATCH // NBUF)
    def _(g):
        for b in range(NBUF):
            jb = g * NBUF + b
            wait_bytes(semg[b], b)
            pltpu.async_copy(rows[b], acc_sh.at[dstb.at[jb]], sems[b],
                             add=True)
            nslot = (b + LEAD) % NBUF
            nj = jb + LEAD

            @pl.when(nj < NBATCH)
            def _():
                @pl.when(nj >= NBUF)
                def _():
                    wait_bytes(sems[nslot], nslot)

                pltpu.async_copy(table_h.at[srcb.at[nj]], rows[nslot],
                                 semg[nslot])

    for b in range(NBUF):
        wait_bytes(sems[b], b)


def _sc_scatter16(table, src2d, dst2d):
    @functools.partial(
        pl.kernel,
        out_type=jax.ShapeDtypeStruct((NC, NPAD, 16), jnp.float32),
        mesh=_mesh,
        scratch_types=[
            pltpu.VMEM_SHARED((NPAD, 16), jnp.float32),
            pltpu.VMEM((NBATCH, EB), jnp.int32),
            pltpu.VMEM((NBATCH, EB), jnp.int32),
            [pltpu.VMEM((EB, 16), jnp.float32)] * NBUF,
            pltpu.VMEM((ZR, 16), jnp.float32),
            [pltpu.SemaphoreType.DMA] * NBUF,
            [pltpu.SemaphoreType.DMA] * NBUF,
        ],
        compiler_params=pltpu.CompilerParams(use_tc_tiling_on_sc=False),
    )
    def k(table_h, src_h, dst_h, out_h, acc_sh, srcb, dstb, rows, zerob,
          semg, sems):
        cid = lax.axis_index("c")
        sid = lax.axis_index("s")
        wid = sid * NC + cid
        _zero_fill(zerob, ZR, 16)
        pltpu.sync_copy(src_h.at[wid, :, :], srcb)
        pltpu.sync_copy(dst_h.at[wid, :, :], dstb)
        sl = pl.ds(sid * ZR, ZR)
        pltpu.sync_copy(zerob, acc_sh.at[sl])
        plsc.subcore_barrier()
        _edge_pipeline(table_h, table_h.at[pl.ds(0, EB)], acc_sh,
                       srcb, dstb, rows, semg, sems)
        plsc.subcore_barrier()
        pltpu.sync_copy(acc_sh.at[sl], out_h.at[cid, sl])

    return k(table, src2d, dst2d)


def _sc_scatter128(vch, src2d, dst2d):
    nch = vch.shape[0]

    @functools.partial(
        pl.kernel,
        out_type=jax.ShapeDtypeStruct((nch, NC, NPAD, FW), jnp.bfloat16),
        mesh=_mesh,
        scratch_types=[
            pltpu.VMEM_SHARED((NPAD, FW), jnp.bfloat16),
            pltpu.VMEM((NBATCH, EB), jnp.int32),
            pltpu.VMEM((NBATCH, EB), jnp.int32),
            [pltpu.VMEM((EB, FW), jnp.bfloat16)] * NBUF,
            pltpu.VMEM((ZR, FW), jnp.bfloat16),
            [pltpu.SemaphoreType.DMA] * NBUF,
            [pltpu.SemaphoreType.DMA] * NBUF,
        ],
        compiler_params=pltpu.CompilerParams(use_tc_tiling_on_sc=False),
    )
    def k(v_h, src_h, dst_h, out_h, acc_sh, srcb, dstb, rows, zerob,
          semg, sems):
        cid = lax.axis_index("c")
        sid = lax.axis_index("s")
        wid = sid * NC + cid
        _zero_fill(zerob, ZR, FW)
        pltpu.sync_copy(src_h.at[wid, :, :], srcb)
        pltpu.sync_copy(dst_h.at[wid, :, :], dstb)
        sl = pl.ds(sid * ZR, ZR)

        @pl.loop(0, nch)
        def _(ch):
            pltpu.sync_copy(zerob, acc_sh.at[sl])
            plsc.subcore_barrier()
            _edge_pipeline(v_h.at[ch], v_h.at[0, pl.ds(0, EB)], acc_sh,
                           srcb, dstb, rows, semg, sems)
            plsc.subcore_barrier()
            pltpu.sync_copy(acc_sh.at[sl], out_h.at[ch, cid, sl])
            plsc.subcore_barrier()

    return k(vch, src2d, dst2d)


def _tc_deg_finalize(h0, h1, xp):
    def body(h0_ref, h1_ref, xp_ref, dis_ref, v1_ref):
        deg = 1.0 + h0_ref[...] + h1_ref[...]
        dis = lax.rsqrt(deg)
        dis_ref[...] = dis
        v1_ref[...] = dis * xp_ref[...]

    return pl.pallas_call(
        body,
        out_shape=(jax.ShapeDtypeStruct((NPAD, 16), jnp.float32),
                   jax.ShapeDtypeStruct((NPAD, 16), jnp.float32)),
    )(h0, h1, xp)


BLK = 512
MGRID = NPAD // BLK


def _tc_mlp_mid(s0, s1, v1, dis, w1p, b1, w2):
    def body(s0_ref, s1_ref, v1_ref, dis_ref, w1_ref, b1_ref, w2_ref, out_ref):
        z = dis_ref[...] * (s0_ref[...] + s1_ref[...] + v1_ref[...])
        h = jnp.dot(z, w1_ref[...], preferred_element_type=jnp.float32)
        h = jnp.maximum(h + b1_ref[...], 0.0)
        u = jnp.dot(h, w2_ref[...], preferred_element_type=jnp.float32)
        v = (dis_ref[...][:, 0:1] * u).astype(jnp.bfloat16)
        for c in range(FCH):
            out_ref[c] = v[:, c * FW:(c + 1) * FW]

    return pl.pallas_call(
        body,
        grid=(MGRID,),
        in_specs=[
            pl.BlockSpec((BLK, 16), lambda i: (i, 0)),
            pl.BlockSpec((BLK, 16), lambda i: (i, 0)),
            pl.BlockSpec((BLK, 16), lambda i: (i, 0)),
            pl.BlockSpec((BLK, 16), lambda i: (i, 0)),
            pl.BlockSpec((16, 2048), lambda i: (0, 0)),
            pl.BlockSpec((1, 2048), lambda i: (0, 0)),
            pl.BlockSpec((2048, 1024), lambda i: (0, 0)),
        ],
        out_specs=pl.BlockSpec((FCH, BLK, FW), lambda i: (0, i, 0)),
        out_shape=jax.ShapeDtypeStruct((FCH, NPAD, FW), jnp.bfloat16),
        compiler_params=pltpu.CompilerParams(
            dimension_semantics=("arbitrary",)),
    )(s0, s1, v1, dis, w1p, b1, w2)


def _tc_pool_half(a0, a1, v, dis, b2, batchp):
    hch = v.shape[0]
    hw = hch * FW

    def body(a0_ref, a1_ref, v_ref, dis_ref, b2_ref, bat_ref,
             out_ref, psum_ref):
        i = pl.program_id(0)

        @pl.when(i == 0)
        def _():
            psum_ref[...] = jnp.zeros_like(psum_ref)

        oh = (bat_ref[...] == lax.broadcasted_iota(jnp.int32, (1, 32), 1))
        oh = oh.astype(jnp.float32)
        dnums = (((0,), (0,)), ((), ()))
        d = dis_ref[...][:, 0:1]
        for c in range(hch):
            y = d * (a0_ref[c, 0].astype(jnp.float32)
                     + a1_ref[c, 0].astype(jnp.float32)
                     + v_ref[c].astype(jnp.float32))
            y = jnp.maximum(y + b2_ref[...][:, c * FW:(c + 1) * FW], 0.0)
            psum_ref[:, c * FW:(c + 1) * FW] += lax.dot_general(
                oh, y, dnums, preferred_element_type=jnp.float32)

        @pl.when(i == MGRID - 1)
        def _():
            out_ref[...] = psum_ref[...]

    return pl.pallas_call(
        body,
        grid=(MGRID,),
        in_specs=[
            pl.BlockSpec((hch, 1, BLK, FW), lambda i: (0, 0, i, 0)),
            pl.BlockSpec((hch, 1, BLK, FW), lambda i: (0, 1, i, 0)),
            pl.BlockSpec((hch, BLK, FW), lambda i: (0, i, 0)),
            pl.BlockSpec((BLK, 16), lambda i: (i, 0)),
            pl.BlockSpec((1, hw), lambda i: (0, 0)),
            pl.BlockSpec((BLK, 1), lambda i: (i, 0)),
        ],
        out_specs=pl.BlockSpec((32, hw), lambda i: (0, 0)),
        out_shape=jax.ShapeDtypeStruct((32, hw), jnp.float32),
        scratch_shapes=[
            pltpu.VMEM((32, hw), jnp.float32),
        ],
        compiler_params=pltpu.CompilerParams(
            dimension_semantics=("arbitrary",)),
    )(a0, a1, v, dis, b2, batchp)


def _tc_head(p0, p1, batchp, wh1, bh1, wh2, bh2, wop, bop):
    def body(p0_ref, p1_ref, bat_ref, wh1_ref, bh1_ref, wh2_ref, bh2_ref,
             wo_ref, bo_ref, out_ref):
        oh = (bat_ref[...] == lax.broadcasted_iota(jnp.int32, (1, 32), 1))
        oh = oh.astype(jnp.float32)
        dnums = (((0,), (0,)), ((), ()))
        cnt = lax.dot_general(oh, jnp.ones((NPAD, 128), jnp.float32), dnums,
                              preferred_element_type=jnp.float32)
        g = jnp.concatenate([p0_ref[...], p1_ref[...]], axis=1)
        g = g / jnp.maximum(cnt[:, 0:1], 1.0)
        g = jnp.dot(g, wh1_ref[...], preferred_element_type=jnp.float32)
        g = jnp.maximum(g + bh1_ref[...], 0.0)
        g = jnp.dot(g, wh2_ref[...], preferred_element_type=jnp.float32)
        g = jnp.maximum(g + bh2_ref[...], 0.0)
        out_ref[...] = (jnp.dot(g, wo_ref[...],
                                preferred_element_type=jnp.float32)
                        + bo_ref[...])

    return pl.pallas_call(
        body,
        out_shape=jax.ShapeDtypeStruct((32, 128), jnp.float32),
    )(p0, p1, batchp, wh1, bh1, wh2, bh2, wop, bop)


def kernel(x, edge_index, batch, W1, b1, W2, b2, Wh1, bh1, Wh2, bh2, Wo, bo):
    f32, i32 = jnp.float32, jnp.int32

    xp = jnp.zeros((NPAD, 16), f32).at[:N_NODES, :5].set(x)
    npad_e = EPAD - E_EDGES
    padidx = DUMMY + jnp.arange(npad_e, dtype=i32) % (NPAD - N_NODES)
    src2d = jnp.concatenate([edge_index[0], padidx]).reshape(NW, NBATCH, EB)
    dst2d = jnp.concatenate([edge_index[1], padidx]).reshape(NW, NBATCH, EB)
    ones16 = jnp.ones((NPAD, 16), f32)
    batchp = jnp.concatenate(
        [batch, jnp.full((NPAD - N_NODES,), 99, i32)]).reshape(NPAD, 1)
    w1p = jnp.zeros((16, 2048), f32).at[:5].set(W1)
    wop = jnp.zeros((512, 128), f32).at[:, :2].set(Wo)
    bop = jnp.zeros((1, 128), f32).at[0, :2].set(bo)

    hist = _sc_scatter16(ones16, dst2d, dst2d)
    dis, v1 = _tc_deg_finalize(hist[0], hist[1], xp)
    s1 = _sc_scatter16(v1, src2d, dst2d)
    v = _tc_mlp_mid(s1[0], s1[1], v1, dis, w1p, b1.reshape(1, -1), W2)
    acc_a = _sc_scatter128(v[:FCH // 2], src2d, dst2d)
    acc_b = _sc_scatter128(v[FCH // 2:], src2d, dst2d)
    b2r = b2.reshape(1, -1)
    p0 = _tc_pool_half(acc_a, acc_a, v[:FCH // 2], dis, b2r[:, :512], batchp)
    p1 = _tc_pool_half(acc_b, acc_b, v[FCH // 2:], dis, b2r[:, 512:], batchp)
    out = _tc_head(p0, p1, batchp, Wh1, bh1.reshape(1, -1), Wh2,
                   bh2.reshape(1, -1), wop, bop)
    return out[:, :2]
